# Initial kernel scaffold; baseline (speedup 1.0000x reference)
import functools

import jax
import jax.numpy as jnp
from jax import lax
from jax.experimental import pallas as pl

N = 10000
E = 320000
F = 128
GATES = 512

ROWS = 1000
GRID = N // ROWS


def _prep_kernel(degp_ref, cntp_ref, x_ref, h_ref,
                 dis_ref, invc_ref, u0x_ref, u0h_ref):
    deg = degp_ref[0, :] + degp_ref[1, :]
    cnt = cntp_ref[0, :] + cntp_ref[1, :]
    dis = jnp.where(deg > 0, lax.rsqrt(jnp.where(deg > 0, deg, 1.0)), 0.0)
    dis_ref[...] = dis[:, None]
    invc_ref[...] = (1.0 / jnp.maximum(cnt, 1.0))[:, None]
    u0x_ref[...] = dis[:, None] * x_ref[...]
    u0h_ref[...] = dis[:, None] * h_ref[...]


def _mid_kernel(s1xp_ref, s1hp_ref, dis_ref,
                tx1x_ref, u1x_ref, tx1h_ref, u1h_ref):
    dis = dis_ref[...]
    s1x = s1xp_ref[0] + s1xp_ref[1]
    s1h = s1hp_ref[0] + s1hp_ref[1]
    tx1x = -dis * s1x
    tx1h = -dis * s1h
    tx1x_ref[...] = tx1x
    tx1h_ref[...] = tx1h
    u1x_ref[...] = dis * tx1x
    u1h_ref[...] = dis * tx1h


def _gates_kernel(x_ref, h_ref, c_ref, tx1x_ref, s2xp_ref, tx1h_ref,
                  s2hp_ref, dis_ref, wx_ref, wh_ref, bias_ref, wcs_ref,
                  hn_ref, cn_ref, sy_ref, sy2_ref):
    i = pl.program_id(0)
    dis = dis_ref[...]
    x = x_ref[...]
    h = h_ref[...]
    c = c_ref[...]
    tx2x = -2.0 * dis * (s2xp_ref[0] + s2xp_ref[1]) - x
    tx2h = -2.0 * dis * (s2hp_ref[0] + s2hp_ref[1]) - h

    def mm(a, w):
        return jnp.dot(a, w, preferred_element_type=jnp.float32)

    z = (mm(x, wx_ref[0]) + mm(tx1x_ref[...], wx_ref[1]) + mm(tx2x, wx_ref[2])
         + mm(h, wh_ref[0]) + mm(tx1h_ref[...], wh_ref[1]) + mm(tx2h, wh_ref[2])
         + bias_ref[...])
    gi = jax.nn.sigmoid(z[:, 0:F] + wcs_ref[0:1, :] * c)
    gf = jax.nn.sigmoid(z[:, F:2 * F] + wcs_ref[1:2, :] * c)
    gt = jnp.tanh(z[:, 2 * F:3 * F])
    cn = gf * c + gi * gt
    go = jax.nn.sigmoid(z[:, 3 * F:4 * F] + wcs_ref[2:3, :] * cn)
    hn = go * jnp.tanh(cn)
    hn_ref[...] = hn
    cn_ref[...] = cn
    y = jnp.maximum(hn, 0.0)

    @pl.when(i == 0)
    def _():
        sy_ref[...] = jnp.zeros_like(sy_ref)
        sy2_ref[...] = jnp.zeros_like(sy2_ref)

    sy_ref[...] += jnp.sum(y, axis=0, keepdims=True)
    sy2_ref[...] += jnp.sum(y * y, axis=0, keepdims=True)


def _proj_kernel(hn_ref, a_ref, shift_ref, wproj_ref, bproj_ref, wr_ref,
                 xp_ref, yr_ref):
    y = a_ref[...] * jnp.maximum(hn_ref[...], 0.0) + shift_ref[...]
    xp = jnp.dot(y, wproj_ref[...], preferred_element_type=jnp.float32)
    xp_ref[...] = jnp.maximum(xp + bproj_ref[...], 0.0)
    yr_ref[...] = jnp.dot(y, wr_ref[...], preferred_element_type=jnp.float32)


def _out_kernel(sp_ref, invc_ref, yr_ref, wl_ref, bl_ref, out_ref):
    mean_nb = (sp_ref[0] + sp_ref[1]) * invc_ref[...]
    out_ref[...] = (jnp.dot(mean_nb, wl_ref[...],
                            preferred_element_type=jnp.float32)
                    + bl_ref[...] + yr_ref[...])


def _row_spec(width):
    return pl.BlockSpec((ROWS, width), lambda i: (i, 0))


def _part_spec(width):
    return pl.BlockSpec((2, ROWS, width), lambda i: (0, i, 0))


def _full_spec(shape):
    return pl.BlockSpec(shape, lambda i: tuple(0 for _ in shape))


def _scatter_partials(v, edge_index):
    out = jnp.zeros((N, v.shape[1]), jnp.float32).at[edge_index[1]].add(
        v[edge_index[0]])
    return jnp.stack([out, jnp.zeros_like(out)])


def kernel(x, edge_index, hidden_state, cell_state, edge_weights, params):
    src, dst = edge_index[0], edge_index[1]

    deg = jnp.zeros((N,), jnp.float32).at[src].add(edge_weights)
    cnt = jnp.zeros((N,), jnp.float32).at[dst].add(1.0)
    degp = jnp.stack([deg, jnp.zeros_like(deg)])
    cntp = jnp.stack([cnt, jnp.zeros_like(cnt)])

    dis, invc, u0x, u0h = pl.pallas_call(
        _prep_kernel,
        grid=(GRID,),
        in_specs=[pl.BlockSpec((2, ROWS), lambda i: (0, i)),
                  pl.BlockSpec((2, ROWS), lambda i: (0, i)),
                  _row_spec(F), _row_spec(F)],
        out_specs=[_row_spec(1), _row_spec(1), _row_spec(F), _row_spec(F)],
        out_shape=[jax.ShapeDtypeStruct((N, 1), jnp.float32),
                   jax.ShapeDtypeStruct((N, 1), jnp.float32),
                   jax.ShapeDtypeStruct((N, F), jnp.float32),
                   jax.ShapeDtypeStruct((N, F), jnp.float32)],
    )(degp, cntp, x, hidden_state)

    s1xp = _scatter_partials(u0x, edge_index)
    s1hp = _scatter_partials(u0h, edge_index)

    tx1x, u1x, tx1h, u1h = pl.pallas_call(
        _mid_kernel,
        grid=(GRID,),
        in_specs=[_part_spec(F), _part_spec(F), _row_spec(1)],
        out_specs=[_row_spec(F)] * 4,
        out_shape=[jax.ShapeDtypeStruct((N, F), jnp.float32)] * 4,
    )(s1xp, s1hp, dis)

    s2xp = _scatter_partials(u1x, edge_index)
    s2hp = _scatter_partials(u1h, edge_index)

    p = params
    wx = jnp.concatenate([p['Wx_i'], p['Wx_f'], p['Wx_c'], p['Wx_o']], axis=2)
    wh = jnp.concatenate([p['Wh_i'], p['Wh_f'], p['Wh_c'], p['Wh_o']], axis=2)
    bias = jnp.concatenate(
        [p['bx_' + g] + p['bh_' + g] + p['b_' + g][0]
         for g in ('i', 'f', 'c', 'o')]).reshape(1, GATES)
    wcs = jnp.concatenate([p['wc_i'], p['wc_f'], p['wc_o']], axis=0)

    hn, cn, sy, sy2 = pl.pallas_call(
        _gates_kernel,
        grid=(GRID,),
        in_specs=[_row_spec(F), _row_spec(F), _row_spec(F), _row_spec(F),
                  _part_spec(F), _row_spec(F), _part_spec(F), _row_spec(1),
                  _full_spec((3, F, GATES)), _full_spec((3, F, GATES)),
                  _full_spec((1, GATES)), _full_spec((3, F))],
        out_specs=[_row_spec(F), _row_spec(F),
                   pl.BlockSpec((1, F), lambda i: (0, 0)),
                   pl.BlockSpec((1, F), lambda i: (0, 0))],
        out_shape=[jax.ShapeDtypeStruct((N, F), jnp.float32),
                   jax.ShapeDtypeStruct((N, F), jnp.float32),
                   jax.ShapeDtypeStruct((1, F), jnp.float32),
                   jax.ShapeDtypeStruct((1, F), jnp.float32)],
    )(x, hidden_state, cell_state, tx1x, s2xp, tx1h, s2hp, dis,
      wx, wh, bias, wcs)

    m = sy / N
    m2 = sy2 / N
    gms = p['gn_mean_scale'][None, :]
    var = m2 - 2.0 * gms * m * m + gms * gms * m * m
    a = p['gn_weight'][None, :] * lax.rsqrt(var + 1e-5)
    shift = p['gn_bias'][None, :] - a * gms * m

    xp, yr = pl.pallas_call(
        _proj_kernel,
        grid=(GRID,),
        in_specs=[_row_spec(F), _full_spec((1, F)), _full_spec((1, F)),
                  _full_spec((F, F)), _full_spec((1, F)), _full_spec((F, 1))],
        out_specs=[_row_spec(F), _row_spec(1)],
        out_shape=[jax.ShapeDtypeStruct((N, F), jnp.float32),
                   jax.ShapeDtypeStruct((N, 1), jnp.float32)],
    )(hn, a, shift, p['W_proj'], p['b_proj'][None, :], p['W_r'])

    sp = _scatter_partials(xp, edge_index)

    out = pl.pallas_call(
        _out_kernel,
        grid=(GRID,),
        in_specs=[_part_spec(F), _row_spec(1), _row_spec(1),
                  _full_spec((F, 1)), _full_spec((1, 1))],
        out_specs=_row_spec(1),
        out_shape=jax.ShapeDtypeStruct((N, 1), jnp.float32),
    )(sp, invc, yr, p['W_l'], p['b_l'][None, :])

    return out, hn, cn


# speedup vs baseline: 1.6317x; 1.6317x over previous
import functools

import jax
import jax.numpy as jnp
from jax import lax
from jax.experimental import pallas as pl

N = 10000
E = 320000
F = 128
GATES = 512

ROWS = 1000
GRID = N // ROWS


def _prep_kernel(degp_ref, cntp_ref, x_ref, h_ref,
                 dis_ref, invc_ref, u0x_ref, u0h_ref):
    deg = degp_ref[0, :, 0] + degp_ref[1, :, 0]
    cnt = cntp_ref[0, :, 0] + cntp_ref[1, :, 0]
    dis = jnp.where(deg > 0, lax.rsqrt(jnp.where(deg > 0, deg, 1.0)), 0.0)
    dis_ref[...] = dis[:, None]
    invc_ref[...] = (1.0 / jnp.maximum(cnt, 1.0))[:, None]
    u0x_ref[...] = dis[:, None] * x_ref[...]
    u0h_ref[...] = dis[:, None] * h_ref[...]


def _mid_kernel(s1xp_ref, s1hp_ref, dis_ref,
                tx1x_ref, u1x_ref, tx1h_ref, u1h_ref):
    dis = dis_ref[...]
    s1x = s1xp_ref[0] + s1xp_ref[1]
    s1h = s1hp_ref[0] + s1hp_ref[1]
    tx1x = -dis * s1x
    tx1h = -dis * s1h
    tx1x_ref[...] = tx1x
    tx1h_ref[...] = tx1h
    u1x_ref[...] = dis * tx1x
    u1h_ref[...] = dis * tx1h


def _gates_kernel(x_ref, h_ref, c_ref, tx1x_ref, s2xp_ref, tx1h_ref,
                  s2hp_ref, dis_ref, wx_ref, wh_ref, bias_ref, wcs_ref,
                  hn_ref, cn_ref, sy_ref, sy2_ref):
    i = pl.program_id(0)
    dis = dis_ref[...]
    x = x_ref[...]
    h = h_ref[...]
    c = c_ref[...]
    tx2x = -2.0 * dis * (s2xp_ref[0] + s2xp_ref[1]) - x
    tx2h = -2.0 * dis * (s2hp_ref[0] + s2hp_ref[1]) - h

    def mm(a, w):
        return jnp.dot(a, w, preferred_element_type=jnp.float32)

    z = (mm(x, wx_ref[0]) + mm(tx1x_ref[...], wx_ref[1]) + mm(tx2x, wx_ref[2])
         + mm(h, wh_ref[0]) + mm(tx1h_ref[...], wh_ref[1]) + mm(tx2h, wh_ref[2])
         + bias_ref[...])
    gi = jax.nn.sigmoid(z[:, 0:F] + wcs_ref[0:1, :] * c)
    gf = jax.nn.sigmoid(z[:, F:2 * F] + wcs_ref[1:2, :] * c)
    gt = jnp.tanh(z[:, 2 * F:3 * F])
    cn = gf * c + gi * gt
    go = jax.nn.sigmoid(z[:, 3 * F:4 * F] + wcs_ref[2:3, :] * cn)
    hn = go * jnp.tanh(cn)
    hn_ref[...] = hn
    cn_ref[...] = cn
    y = jnp.maximum(hn, 0.0)

    @pl.when(i == 0)
    def _():
        sy_ref[...] = jnp.zeros_like(sy_ref)
        sy2_ref[...] = jnp.zeros_like(sy2_ref)

    sy_ref[...] += jnp.sum(y, axis=0, keepdims=True)
    sy2_ref[...] += jnp.sum(y * y, axis=0, keepdims=True)


def _proj_kernel(hn_ref, a_ref, shift_ref, wproj_ref, bproj_ref, wr_ref,
                 xp_ref, yr_ref):
    y = a_ref[...] * jnp.maximum(hn_ref[...], 0.0) + shift_ref[...]
    xp = jnp.dot(y, wproj_ref[...], preferred_element_type=jnp.float32)
    xp_ref[...] = jnp.maximum(xp + bproj_ref[...], 0.0)
    yr_ref[...] = jnp.dot(y, wr_ref[...], preferred_element_type=jnp.float32)


def _out_kernel(sp_ref, invc_ref, yr_ref, wl_ref, bl_ref, out_ref):
    mean_nb = (sp_ref[0] + sp_ref[1]) * invc_ref[...]
    out_ref[...] = (jnp.dot(mean_nb, wl_ref[...],
                            preferred_element_type=jnp.float32)
                    + bl_ref[...] + yr_ref[...])


def _row_spec(width):
    return pl.BlockSpec((ROWS, width), lambda i: (i, 0))


def _part_spec(width):
    return pl.BlockSpec((2, ROWS, width), lambda i: (0, i, 0))


def _full_spec(shape):
    return pl.BlockSpec(shape, lambda i: tuple(0 for _ in shape))


def _scatter_partials(v, edge_index):
    out = jnp.zeros((N, v.shape[1]), jnp.float32).at[edge_index[1]].add(
        v[edge_index[0]])
    return jnp.stack([out, jnp.zeros_like(out)])


def kernel(x, edge_index, hidden_state, cell_state, edge_weights, params):
    src, dst = edge_index[0], edge_index[1]

    deg = jnp.zeros((N,), jnp.float32).at[src].add(edge_weights)
    cnt = jnp.zeros((N,), jnp.float32).at[dst].add(1.0)
    degp = jnp.stack([deg, jnp.zeros_like(deg)])[:, :, None]
    cntp = jnp.stack([cnt, jnp.zeros_like(cnt)])[:, :, None]

    dis, invc, u0x, u0h = pl.pallas_call(
        _prep_kernel,
        grid=(GRID,),
        in_specs=[pl.BlockSpec((2, ROWS, 1), lambda i: (0, i, 0)),
                  pl.BlockSpec((2, ROWS, 1), lambda i: (0, i, 0)),
                  _row_spec(F), _row_spec(F)],
        out_specs=[_row_spec(1), _row_spec(1), _row_spec(F), _row_spec(F)],
        out_shape=[jax.ShapeDtypeStruct((N, 1), jnp.float32),
                   jax.ShapeDtypeStruct((N, 1), jnp.float32),
                   jax.ShapeDtypeStruct((N, F), jnp.float32),
                   jax.ShapeDtypeStruct((N, F), jnp.float32)],
    )(degp, cntp, x, hidden_state)

    s1xp = _scatter_partials(u0x, edge_index)
    s1hp = _scatter_partials(u0h, edge_index)

    tx1x, u1x, tx1h, u1h = pl.pallas_call(
        _mid_kernel,
        grid=(GRID,),
        in_specs=[_part_spec(F), _part_spec(F), _row_spec(1)],
        out_specs=[_row_spec(F)] * 4,
        out_shape=[jax.ShapeDtypeStruct((N, F), jnp.float32)] * 4,
    )(s1xp, s1hp, dis)

    s2xp = _scatter_partials(u1x, edge_index)
    s2hp = _scatter_partials(u1h, edge_index)

    p = params
    wx = jnp.concatenate([p['Wx_i'], p['Wx_f'], p['Wx_c'], p['Wx_o']], axis=2)
    wh = jnp.concatenate([p['Wh_i'], p['Wh_f'], p['Wh_c'], p['Wh_o']], axis=2)
    bias = jnp.concatenate(
        [p['bx_' + g] + p['bh_' + g] + p['b_' + g][0]
         for g in ('i', 'f', 'c', 'o')]).reshape(1, GATES)
    wcs = jnp.concatenate([p['wc_i'], p['wc_f'], p['wc_o']], axis=0)

    hn, cn, sy, sy2 = pl.pallas_call(
        _gates_kernel,
        grid=(GRID,),
        in_specs=[_row_spec(F), _row_spec(F), _row_spec(F), _row_spec(F),
                  _part_spec(F), _row_spec(F), _part_spec(F), _row_spec(1),
                  _full_spec((3, F, GATES)), _full_spec((3, F, GATES)),
                  _full_spec((1, GATES)), _full_spec((3, F))],
        out_specs=[_row_spec(F), _row_spec(F),
                   pl.BlockSpec((1, F), lambda i: (0, 0)),
                   pl.BlockSpec((1, F), lambda i: (0, 0))],
        out_shape=[jax.ShapeDtypeStruct((N, F), jnp.float32),
                   jax.ShapeDtypeStruct((N, F), jnp.float32),
                   jax.ShapeDtypeStruct((1, F), jnp.float32),
                   jax.ShapeDtypeStruct((1, F), jnp.float32)],
    )(x, hidden_state, cell_state, tx1x, s2xp, tx1h, s2hp, dis,
      wx, wh, bias, wcs)

    m = sy / N
    m2 = sy2 / N
    gms = p['gn_mean_scale'][None, :]
    var = m2 - 2.0 * gms * m * m + gms * gms * m * m
    a = p['gn_weight'][None, :] * lax.rsqrt(var + 1e-5)
    shift = p['gn_bias'][None, :] - a * gms * m

    xp, yr = pl.pallas_call(
        _proj_kernel,
        grid=(GRID,),
        in_specs=[_row_spec(F), _full_spec((1, F)), _full_spec((1, F)),
                  _full_spec((F, F)), _full_spec((1, F)), _full_spec((F, 1))],
        out_specs=[_row_spec(F), _row_spec(1)],
        out_shape=[jax.ShapeDtypeStruct((N, F), jnp.float32),
                   jax.ShapeDtypeStruct((N, 1), jnp.float32)],
    )(hn, a, shift, p['W_proj'], p['b_proj'][None, :], p['W_r'])

    sp = _scatter_partials(xp, edge_index)

    out = pl.pallas_call(
        _out_kernel,
        grid=(GRID,),
        in_specs=[_part_spec(F), _row_spec(1), _row_spec(1),
                  _full_spec((F, 1)), _full_spec((1, 1))],
        out_specs=_row_spec(1),
        out_shape=jax.ShapeDtypeStruct((N, 1), jnp.float32),
    )(sp, invc, yr, p['W_l'], p['b_l'][None, :])

    return out, hn, cn


# speedup vs baseline: 7.0438x; 4.3167x over previous
import functools

import jax
import jax.numpy as jnp
from jax import lax
from jax.experimental import pallas as pl
from jax.experimental.pallas import tpu as pltpu
from jax.experimental.pallas import tpu_sc as plsc

N = 10000
E = 320000
F = 128
GATES = 512

ROWS = 1000
GRID = N // ROWS

NC = 2
NS = 16
NW = NC * NS
EB = 128
NBLK = E // EB
WITER = (NBLK + NW - 1) // NW
DCH = 80
NCH = N // DCH
NP = 10240
DCH1 = 128
NCH1 = NP // DCH1


def _sc_mesh():
    return plsc.VectorSubcoreMesh(core_axis_name="c", subcore_axis_name="s")


def _zero_acc(sid, zeros_hbm, acc, dch, nch):
    citer = (nch + NS - 1) // NS

    def zbody(k, carry):
        ch = sid + k * NS

        @pl.when(ch < nch)
        def _():
            sl = pl.ds(ch * dch, dch)
            pltpu.sync_copy(zeros_hbm.at[sl], acc.at[sl])
        return carry

    lax.fori_loop(0, citer, zbody, 0)


def _dump_acc(cid, sid, acc, bounce, out_hbm, dch, nch):
    citer = (nch + NS - 1) // NS

    def dbody(k, carry):
        ch = sid + k * NS

        @pl.when(ch < nch)
        def _():
            sl = pl.ds(ch * dch, dch)
            pltpu.sync_copy(acc.at[sl], bounce)
            pltpu.sync_copy(bounce, out_hbm.at[cid].at[sl])
        return carry

    lax.fori_loop(0, citer, dbody, 0)


def _spmv_phase(cid, sid, v_hbm, src_hbm, dst_hbm, zeros_hbm, out_hbm,
                idxs, idxd, rows, bounce, acc, sem):
    wid = cid * NS + sid
    _zero_acc(sid, zeros_hbm, acc, DCH, NCH)
    plsc.subcore_barrier()

    def ebody(i, carry):
        b = wid + i * NW

        @pl.when(b < NBLK)
        def _():
            base = b * EB
            pltpu.sync_copy(src_hbm.at[pl.ds(base, EB)], idxs)
            pltpu.sync_copy(dst_hbm.at[pl.ds(base, EB)], idxd)
            pltpu.async_copy(v_hbm.at[idxs], rows, sem).wait()
            pltpu.sync_copy(rows, acc.at[idxd], add=True)
        return carry

    lax.fori_loop(0, WITER, ebody, 0)
    plsc.subcore_barrier()
    _dump_acc(cid, sid, acc, bounce, out_hbm, DCH, NCH)


def _make_spmv(num_phases):
    @functools.partial(
        pl.kernel,
        mesh=_sc_mesh(),
        out_type=[jax.ShapeDtypeStruct((NC, N, F), jnp.float32)] * num_phases,
        scratch_types=[
            pltpu.VMEM((EB,), jnp.int32),
            pltpu.VMEM((EB,), jnp.int32),
            pltpu.VMEM((EB, F), jnp.float32),
            pltpu.VMEM((DCH, F), jnp.float32),
            pltpu.VMEM_SHARED((N, F), jnp.float32),
            pltpu.SemaphoreType.DMA,
        ],
    )
    def spmv(*refs):
        vs = refs[:num_phases]
        src_hbm, dst_hbm, zeros_hbm = refs[num_phases:num_phases + 3]
        outs = refs[num_phases + 3:num_phases + 3 + num_phases]
        idxs, idxd, rows, bounce, acc, sem = refs[num_phases + 3 + num_phases:]
        cid = lax.axis_index("c")
        sid = lax.axis_index("s")
        for v_hbm, out_hbm in zip(vs, outs):
            _spmv_phase(cid, sid, v_hbm, src_hbm, dst_hbm, zeros_hbm,
                        out_hbm, idxs, idxd, rows, bounce, acc, sem)
            plsc.subcore_barrier()

    return spmv


@functools.partial(
    pl.kernel,
    mesh=_sc_mesh(),
    out_type=[jax.ShapeDtypeStruct((NC, NP), jnp.float32)] * 2,
    scratch_types=[
        pltpu.VMEM((EB,), jnp.int32),
        pltpu.VMEM((EB,), jnp.int32),
        pltpu.VMEM((EB,), jnp.float32),
        pltpu.VMEM((DCH1,), jnp.float32),
        pltpu.VMEM_SHARED((NP,), jnp.float32),
        pltpu.VMEM_SHARED((NP,), jnp.float32),
    ],
)
def _hist_sc(src_hbm, dst_hbm, zeros_hbm, deg_hbm, cnt_hbm,
             idxs, idxd, onesb, bounce, accd, accc):
    cid = lax.axis_index("c")
    sid = lax.axis_index("s")
    wid = cid * NS + sid

    def obody(j, carry):
        onesb[pl.ds(j * 16, 16)] = jnp.ones((16,), jnp.float32)
        return carry

    lax.fori_loop(0, EB // 16, obody, 0)
    _zero_acc(sid, zeros_hbm, accd, DCH1, NCH1)
    _zero_acc(sid, zeros_hbm, accc, DCH1, NCH1)
    plsc.subcore_barrier()

    def ebody(i, carry):
        b = wid + i * NW

        @pl.when(b < NBLK)
        def _():
            base = b * EB
            pltpu.sync_copy(src_hbm.at[pl.ds(base, EB)], idxs)
            pltpu.sync_copy(dst_hbm.at[pl.ds(base, EB)], idxd)
            pltpu.sync_copy(onesb, accd.at[idxs], add=True)
            pltpu.sync_copy(onesb, accc.at[idxd], add=True)
        return carry

    lax.fori_loop(0, WITER, ebody, 0)
    plsc.subcore_barrier()
    _dump_acc(cid, sid, accd, bounce, deg_hbm, DCH1, NCH1)
    _dump_acc(cid, sid, accc, bounce, cnt_hbm, DCH1, NCH1)


_SPMV2 = _make_spmv(2)
_SPMV1 = _make_spmv(1)


def _prep_kernel(degp_ref, cntp_ref, x_ref, h_ref,
                 dis_ref, invc_ref, u0x_ref, u0h_ref):
    deg = degp_ref[0, :, 0] + degp_ref[1, :, 0]
    cnt = cntp_ref[0, :, 0] + cntp_ref[1, :, 0]
    dis = jnp.where(deg > 0, lax.rsqrt(jnp.where(deg > 0, deg, 1.0)), 0.0)
    dis_ref[...] = dis[:, None]
    invc_ref[...] = (1.0 / jnp.maximum(cnt, 1.0))[:, None]
    u0x_ref[...] = dis[:, None] * x_ref[...]
    u0h_ref[...] = dis[:, None] * h_ref[...]


def _mid_kernel(s1xp_ref, s1hp_ref, dis_ref,
                tx1x_ref, u1x_ref, tx1h_ref, u1h_ref):
    dis = dis_ref[...]
    s1x = s1xp_ref[0] + s1xp_ref[1]
    s1h = s1hp_ref[0] + s1hp_ref[1]
    tx1x = -dis * s1x
    tx1h = -dis * s1h
    tx1x_ref[...] = tx1x
    tx1h_ref[...] = tx1h
    u1x_ref[...] = dis * tx1x
    u1h_ref[...] = dis * tx1h


def _gates_kernel(x_ref, h_ref, c_ref, tx1x_ref, s2xp_ref, tx1h_ref,
                  s2hp_ref, dis_ref, wx_ref, wh_ref, bias_ref, wcs_ref,
                  hn_ref, cn_ref, sy_ref, sy2_ref):
    i = pl.program_id(0)
    dis = dis_ref[...]
    x = x_ref[...]
    h = h_ref[...]
    c = c_ref[...]
    tx2x = -2.0 * dis * (s2xp_ref[0] + s2xp_ref[1]) - x
    tx2h = -2.0 * dis * (s2hp_ref[0] + s2hp_ref[1]) - h

    def mm(a, w):
        return jnp.dot(a, w, preferred_element_type=jnp.float32)

    z = (mm(x, wx_ref[0]) + mm(tx1x_ref[...], wx_ref[1]) + mm(tx2x, wx_ref[2])
         + mm(h, wh_ref[0]) + mm(tx1h_ref[...], wh_ref[1]) + mm(tx2h, wh_ref[2])
         + bias_ref[...])
    gi = jax.nn.sigmoid(z[:, 0:F] + wcs_ref[0:1, :] * c)
    gf = jax.nn.sigmoid(z[:, F:2 * F] + wcs_ref[1:2, :] * c)
    gt = jnp.tanh(z[:, 2 * F:3 * F])
    cn = gf * c + gi * gt
    go = jax.nn.sigmoid(z[:, 3 * F:4 * F] + wcs_ref[2:3, :] * cn)
    hn = go * jnp.tanh(cn)
    hn_ref[...] = hn
    cn_ref[...] = cn
    y = jnp.maximum(hn, 0.0)

    @pl.when(i == 0)
    def _():
        sy_ref[...] = jnp.zeros_like(sy_ref)
        sy2_ref[...] = jnp.zeros_like(sy2_ref)

    sy_ref[...] += jnp.sum(y, axis=0, keepdims=True)
    sy2_ref[...] += jnp.sum(y * y, axis=0, keepdims=True)


def _proj_kernel(hn_ref, a_ref, shift_ref, wproj_ref, bproj_ref, wr_ref,
                 xp_ref, yr_ref):
    y = a_ref[...] * jnp.maximum(hn_ref[...], 0.0) + shift_ref[...]
    xp = jnp.dot(y, wproj_ref[...], preferred_element_type=jnp.float32)
    xp_ref[...] = jnp.maximum(xp + bproj_ref[...], 0.0)
    yr_ref[...] = jnp.dot(y, wr_ref[...], preferred_element_type=jnp.float32)


def _out_kernel(sp_ref, invc_ref, yr_ref, wl_ref, bl_ref, out_ref):
    mean_nb = (sp_ref[0] + sp_ref[1]) * invc_ref[...]
    out_ref[...] = (jnp.dot(mean_nb, wl_ref[...],
                            preferred_element_type=jnp.float32)
                    + bl_ref[...] + yr_ref[...])


def _row_spec(width):
    return pl.BlockSpec((ROWS, width), lambda i: (i, 0))


def _part_spec(width):
    return pl.BlockSpec((2, ROWS, width), lambda i: (0, i, 0))


def _full_spec(shape):
    return pl.BlockSpec(shape, lambda i: tuple(0 for _ in shape))


def kernel(x, edge_index, hidden_state, cell_state, edge_weights, params):
    src, dst = edge_index[0], edge_index[1]
    zeros_nf = jnp.zeros((N, F), jnp.float32)
    zeros_np = jnp.zeros((NP,), jnp.float32)

    degp2, cntp2 = _hist_sc(src, dst, zeros_np)
    degp = degp2[:, :N, None]
    cntp = cntp2[:, :N, None]

    dis, invc, u0x, u0h = pl.pallas_call(
        _prep_kernel,
        grid=(GRID,),
        in_specs=[pl.BlockSpec((2, ROWS, 1), lambda i: (0, i, 0)),
                  pl.BlockSpec((2, ROWS, 1), lambda i: (0, i, 0)),
                  _row_spec(F), _row_spec(F)],
        out_specs=[_row_spec(1), _row_spec(1), _row_spec(F), _row_spec(F)],
        out_shape=[jax.ShapeDtypeStruct((N, 1), jnp.float32),
                   jax.ShapeDtypeStruct((N, 1), jnp.float32),
                   jax.ShapeDtypeStruct((N, F), jnp.float32),
                   jax.ShapeDtypeStruct((N, F), jnp.float32)],
    )(degp, cntp, x, hidden_state)

    s1xp, s1hp = _SPMV2(u0x, u0h, src, dst, zeros_nf)

    tx1x, u1x, tx1h, u1h = pl.pallas_call(
        _mid_kernel,
        grid=(GRID,),
        in_specs=[_part_spec(F), _part_spec(F), _row_spec(1)],
        out_specs=[_row_spec(F)] * 4,
        out_shape=[jax.ShapeDtypeStruct((N, F), jnp.float32)] * 4,
    )(s1xp, s1hp, dis)

    s2xp, s2hp = _SPMV2(u1x, u1h, src, dst, zeros_nf)

    p = params
    wx = jnp.concatenate([p['Wx_i'], p['Wx_f'], p['Wx_c'], p['Wx_o']], axis=2)
    wh = jnp.concatenate([p['Wh_i'], p['Wh_f'], p['Wh_c'], p['Wh_o']], axis=2)
    bias = jnp.concatenate(
        [p['bx_' + g] + p['bh_' + g] + p['b_' + g][0]
         for g in ('i', 'f', 'c', 'o')]).reshape(1, GATES)
    wcs = jnp.concatenate([p['wc_i'], p['wc_f'], p['wc_o']], axis=0)

    hn, cn, sy, sy2 = pl.pallas_call(
        _gates_kernel,
        grid=(GRID,),
        in_specs=[_row_spec(F), _row_spec(F), _row_spec(F), _row_spec(F),
                  _part_spec(F), _row_spec(F), _part_spec(F), _row_spec(1),
                  _full_spec((3, F, GATES)), _full_spec((3, F, GATES)),
                  _full_spec((1, GATES)), _full_spec((3, F))],
        out_specs=[_row_spec(F), _row_spec(F),
                   pl.BlockSpec((1, F), lambda i: (0, 0)),
                   pl.BlockSpec((1, F), lambda i: (0, 0))],
        out_shape=[jax.ShapeDtypeStruct((N, F), jnp.float32),
                   jax.ShapeDtypeStruct((N, F), jnp.float32),
                   jax.ShapeDtypeStruct((1, F), jnp.float32),
                   jax.ShapeDtypeStruct((1, F), jnp.float32)],
    )(x, hidden_state, cell_state, tx1x, s2xp, tx1h, s2hp, dis,
      wx, wh, bias, wcs)

    m = sy / N
    m2 = sy2 / N
    gms = p['gn_mean_scale'][None, :]
    var = m2 - 2.0 * gms * m * m + gms * gms * m * m
    a = p['gn_weight'][None, :] * lax.rsqrt(var + 1e-5)
    shift = p['gn_bias'][None, :] - a * gms * m

    xp, yr = pl.pallas_call(
        _proj_kernel,
        grid=(GRID,),
        in_specs=[_row_spec(F), _full_spec((1, F)), _full_spec((1, F)),
                  _full_spec((F, F)), _full_spec((1, F)), _full_spec((F, 1))],
        out_specs=[_row_spec(F), _row_spec(1)],
        out_shape=[jax.ShapeDtypeStruct((N, F), jnp.float32),
                   jax.ShapeDtypeStruct((N, 1), jnp.float32)],
    )(hn, a, shift, p['W_proj'], p['b_proj'][None, :], p['W_r'])

    (sp,) = _SPMV1(xp, src, dst, zeros_nf)

    out = pl.pallas_call(
        _out_kernel,
        grid=(GRID,),
        in_specs=[_part_spec(F), _row_spec(1), _row_spec(1),
                  _full_spec((F, 1)), _full_spec((1, 1))],
        out_specs=_row_spec(1),
        out_shape=jax.ShapeDtypeStruct((N, 1), jnp.float32),
    )(sp, invc, yr, p['W_l'], p['b_l'][None, :])

    return out, hn, cn


# speedup vs baseline: 11.5955x; 1.6462x over previous
import functools

import jax
import jax.numpy as jnp
from jax import lax
from jax.experimental import pallas as pl
from jax.experimental.pallas import tpu as pltpu
from jax.experimental.pallas import tpu_sc as plsc

N = 10000
E = 320000
F = 128
GATES = 512

ROWS = 1000
GRID = N // ROWS

NC = 2
NS = 16
NW = NC * NS
EB = 128
NBLK = E // EB
WITER = (NBLK + NW - 1) // NW
DCH = 80
NCH = N // DCH
NP = 10240
DCH1 = 128
NCH1 = NP // DCH1


def _sc_mesh():
    return plsc.VectorSubcoreMesh(core_axis_name="c", subcore_axis_name="s")


def _zero_acc(sid, zeros_hbm, acc, dch, nch):
    citer = (nch + NS - 1) // NS

    def zbody(k, carry):
        ch = sid + k * NS

        @pl.when(ch < nch)
        def _():
            sl = pl.ds(ch * dch, dch)
            pltpu.sync_copy(zeros_hbm.at[sl], acc.at[sl])
        return carry

    lax.fori_loop(0, citer, zbody, 0)


def _dump_acc(cid, sid, acc, bounce, out_hbm, dch, nch):
    citer = (nch + NS - 1) // NS

    def dbody(k, carry):
        ch = sid + k * NS

        @pl.when(ch < nch)
        def _():
            sl = pl.ds(ch * dch, dch)
            pltpu.sync_copy(acc.at[sl], bounce)
            pltpu.sync_copy(bounce, out_hbm.at[cid].at[sl])
        return carry

    lax.fori_loop(0, citer, dbody, 0)


def _spmv_phase(cid, sid, v_hbm, src_hbm, dst_hbm, zeros_hbm, out_hbm,
                bufs, bounce, acc):
    wid = cid * NS + sid
    _zero_acc(sid, zeros_hbm, acc, DCH, NCH)
    plsc.subcore_barrier()

    def valid(i):
        return wid + i * NW < NBLK

    def base(i):
        return (wid + i * NW) * EB

    def idx_copies(i, buf):
        idxs, idxd = buf[0], buf[1]
        semi = buf[4]
        return (pltpu.make_async_copy(src_hbm.at[pl.ds(base(i), EB)],
                                      idxs, semi),
                pltpu.make_async_copy(dst_hbm.at[pl.ds(base(i), EB)],
                                      idxd, semi))

    def gather_copy(buf):
        return pltpu.make_async_copy(v_hbm.at[buf[0]], buf[2], buf[3])

    @pl.when(valid(0))
    def _():
        c0, c1 = idx_copies(0, bufs[0])
        c0.start()
        c1.start()
        c0.wait()
        c1.wait()
        gather_copy(bufs[0]).start()

    @pl.when(valid(1))
    def _():
        c0, c1 = idx_copies(1, bufs[1])
        c0.start()
        c1.start()

    def half(i, bp, bq):
        @pl.when(valid(i))
        def _():
            gather_copy(bp).wait()

            @pl.when(valid(i + 1))
            def _():
                c0, c1 = idx_copies(i + 1, bq)
                c0.wait()
                c1.wait()
                gather_copy(bq).start()

            pltpu.sync_copy(bp[2], acc.at[bp[1]], add=True)

            @pl.when(valid(i + 2))
            def _():
                c0, c1 = idx_copies(i + 2, bp)
                c0.start()
                c1.start()

    def ebody(j, carry):
        half(2 * j, bufs[0], bufs[1])
        half(2 * j + 1, bufs[1], bufs[0])
        return carry

    lax.fori_loop(0, (WITER + 1) // 2, ebody, 0)
    plsc.subcore_barrier()
    _dump_acc(cid, sid, acc, bounce, out_hbm, DCH, NCH)


def _make_spmv(num_phases):
    @functools.partial(
        pl.kernel,
        mesh=_sc_mesh(),
        out_type=[jax.ShapeDtypeStruct((NC, N, F), jnp.float32)] * num_phases,
        scratch_types=[
            pltpu.VMEM((EB,), jnp.int32),
            pltpu.VMEM((EB,), jnp.int32),
            pltpu.VMEM((EB, F), jnp.float32),
            pltpu.SemaphoreType.DMA,
            pltpu.SemaphoreType.DMA,
            pltpu.VMEM((EB,), jnp.int32),
            pltpu.VMEM((EB,), jnp.int32),
            pltpu.VMEM((EB, F), jnp.float32),
            pltpu.SemaphoreType.DMA,
            pltpu.SemaphoreType.DMA,
            pltpu.VMEM((DCH, F), jnp.float32),
            pltpu.VMEM_SHARED((N, F), jnp.float32),
        ],
    )
    def spmv(*refs):
        vs = refs[:num_phases]
        src_hbm, dst_hbm, zeros_hbm = refs[num_phases:num_phases + 3]
        outs = refs[num_phases + 3:num_phases + 3 + num_phases]
        scr = refs[num_phases + 3 + num_phases:]
        bufs = (scr[0:5], scr[5:10])
        bounce, acc = scr[10], scr[11]
        cid = lax.axis_index("c")
        sid = lax.axis_index("s")
        for v_hbm, out_hbm in zip(vs, outs):
            _spmv_phase(cid, sid, v_hbm, src_hbm, dst_hbm, zeros_hbm,
                        out_hbm, bufs, bounce, acc)
            plsc.subcore_barrier()

    return spmv


@functools.partial(
    pl.kernel,
    mesh=_sc_mesh(),
    out_type=[jax.ShapeDtypeStruct((NC, NP), jnp.float32)] * 2,
    scratch_types=[
        pltpu.VMEM((EB,), jnp.int32),
        pltpu.VMEM((EB,), jnp.int32),
        pltpu.VMEM((EB,), jnp.float32),
        pltpu.VMEM((DCH1,), jnp.float32),
        pltpu.VMEM_SHARED((NP,), jnp.float32),
        pltpu.VMEM_SHARED((NP,), jnp.float32),
    ],
)
def _hist_sc(src_hbm, dst_hbm, zeros_hbm, deg_hbm, cnt_hbm,
             idxs, idxd, onesb, bounce, accd, accc):
    cid = lax.axis_index("c")
    sid = lax.axis_index("s")
    wid = cid * NS + sid

    def obody(j, carry):
        onesb[pl.ds(j * 16, 16)] = jnp.ones((16,), jnp.float32)
        return carry

    lax.fori_loop(0, EB // 16, obody, 0)
    _zero_acc(sid, zeros_hbm, accd, DCH1, NCH1)
    _zero_acc(sid, zeros_hbm, accc, DCH1, NCH1)
    plsc.subcore_barrier()

    def ebody(i, carry):
        b = wid + i * NW

        @pl.when(b < NBLK)
        def _():
            base = b * EB
            pltpu.sync_copy(src_hbm.at[pl.ds(base, EB)], idxs)
            pltpu.sync_copy(dst_hbm.at[pl.ds(base, EB)], idxd)
            pltpu.sync_copy(onesb, accd.at[idxs], add=True)
            pltpu.sync_copy(onesb, accc.at[idxd], add=True)
        return carry

    lax.fori_loop(0, WITER, ebody, 0)
    plsc.subcore_barrier()
    _dump_acc(cid, sid, accd, bounce, deg_hbm, DCH1, NCH1)
    _dump_acc(cid, sid, accc, bounce, cnt_hbm, DCH1, NCH1)


_SPMV2 = _make_spmv(2)
_SPMV1 = _make_spmv(1)


def _prep_kernel(degp_ref, cntp_ref, x_ref, h_ref,
                 dis_ref, invc_ref, u0x_ref, u0h_ref):
    deg = degp_ref[0, :, 0] + degp_ref[1, :, 0]
    cnt = cntp_ref[0, :, 0] + cntp_ref[1, :, 0]
    dis = jnp.where(deg > 0, lax.rsqrt(jnp.where(deg > 0, deg, 1.0)), 0.0)
    dis_ref[...] = dis[:, None]
    invc_ref[...] = (1.0 / jnp.maximum(cnt, 1.0))[:, None]
    u0x_ref[...] = dis[:, None] * x_ref[...]
    u0h_ref[...] = dis[:, None] * h_ref[...]


def _mid_kernel(s1xp_ref, s1hp_ref, dis_ref,
                tx1x_ref, u1x_ref, tx1h_ref, u1h_ref):
    dis = dis_ref[...]
    s1x = s1xp_ref[0] + s1xp_ref[1]
    s1h = s1hp_ref[0] + s1hp_ref[1]
    tx1x = -dis * s1x
    tx1h = -dis * s1h
    tx1x_ref[...] = tx1x
    tx1h_ref[...] = tx1h
    u1x_ref[...] = dis * tx1x
    u1h_ref[...] = dis * tx1h


def _gates_kernel(x_ref, h_ref, c_ref, tx1x_ref, s2xp_ref, tx1h_ref,
                  s2hp_ref, dis_ref, wx_ref, wh_ref, bias_ref, wcs_ref,
                  hn_ref, cn_ref, sy_ref, sy2_ref):
    i = pl.program_id(0)
    dis = dis_ref[...]
    x = x_ref[...]
    h = h_ref[...]
    c = c_ref[...]
    tx2x = -2.0 * dis * (s2xp_ref[0] + s2xp_ref[1]) - x
    tx2h = -2.0 * dis * (s2hp_ref[0] + s2hp_ref[1]) - h

    def mm(a, w):
        return jnp.dot(a, w, preferred_element_type=jnp.float32)

    z = (mm(x, wx_ref[0]) + mm(tx1x_ref[...], wx_ref[1]) + mm(tx2x, wx_ref[2])
         + mm(h, wh_ref[0]) + mm(tx1h_ref[...], wh_ref[1]) + mm(tx2h, wh_ref[2])
         + bias_ref[...])
    gi = jax.nn.sigmoid(z[:, 0:F] + wcs_ref[0:1, :] * c)
    gf = jax.nn.sigmoid(z[:, F:2 * F] + wcs_ref[1:2, :] * c)
    gt = jnp.tanh(z[:, 2 * F:3 * F])
    cn = gf * c + gi * gt
    go = jax.nn.sigmoid(z[:, 3 * F:4 * F] + wcs_ref[2:3, :] * cn)
    hn = go * jnp.tanh(cn)
    hn_ref[...] = hn
    cn_ref[...] = cn
    y = jnp.maximum(hn, 0.0)

    @pl.when(i == 0)
    def _():
        sy_ref[...] = jnp.zeros_like(sy_ref)
        sy2_ref[...] = jnp.zeros_like(sy2_ref)

    sy_ref[...] += jnp.sum(y, axis=0, keepdims=True)
    sy2_ref[...] += jnp.sum(y * y, axis=0, keepdims=True)


def _proj_kernel(hn_ref, a_ref, shift_ref, wproj_ref, bproj_ref, wr_ref,
                 xp_ref, yr_ref):
    y = a_ref[...] * jnp.maximum(hn_ref[...], 0.0) + shift_ref[...]
    xp = jnp.dot(y, wproj_ref[...], preferred_element_type=jnp.float32)
    xp_ref[...] = jnp.maximum(xp + bproj_ref[...], 0.0)
    yr_ref[...] = jnp.dot(y, wr_ref[...], preferred_element_type=jnp.float32)


def _out_kernel(sp_ref, invc_ref, yr_ref, wl_ref, bl_ref, out_ref):
    mean_nb = (sp_ref[0] + sp_ref[1]) * invc_ref[...]
    out_ref[...] = (jnp.dot(mean_nb, wl_ref[...],
                            preferred_element_type=jnp.float32)
                    + bl_ref[...] + yr_ref[...])


def _row_spec(width):
    return pl.BlockSpec((ROWS, width), lambda i: (i, 0))


def _part_spec(width):
    return pl.BlockSpec((2, ROWS, width), lambda i: (0, i, 0))


def _full_spec(shape):
    return pl.BlockSpec(shape, lambda i: tuple(0 for _ in shape))


def kernel(x, edge_index, hidden_state, cell_state, edge_weights, params):
    src, dst = edge_index[0], edge_index[1]
    zeros_nf = jnp.zeros((N, F), jnp.float32)
    zeros_np = jnp.zeros((NP,), jnp.float32)

    degp2, cntp2 = _hist_sc(src, dst, zeros_np)
    degp = degp2[:, :N, None]
    cntp = cntp2[:, :N, None]

    dis, invc, u0x, u0h = pl.pallas_call(
        _prep_kernel,
        grid=(GRID,),
        in_specs=[pl.BlockSpec((2, ROWS, 1), lambda i: (0, i, 0)),
                  pl.BlockSpec((2, ROWS, 1), lambda i: (0, i, 0)),
                  _row_spec(F), _row_spec(F)],
        out_specs=[_row_spec(1), _row_spec(1), _row_spec(F), _row_spec(F)],
        out_shape=[jax.ShapeDtypeStruct((N, 1), jnp.float32),
                   jax.ShapeDtypeStruct((N, 1), jnp.float32),
                   jax.ShapeDtypeStruct((N, F), jnp.float32),
                   jax.ShapeDtypeStruct((N, F), jnp.float32)],
    )(degp, cntp, x, hidden_state)

    s1xp, s1hp = _SPMV2(u0x, u0h, src, dst, zeros_nf)

    tx1x, u1x, tx1h, u1h = pl.pallas_call(
        _mid_kernel,
        grid=(GRID,),
        in_specs=[_part_spec(F), _part_spec(F), _row_spec(1)],
        out_specs=[_row_spec(F)] * 4,
        out_shape=[jax.ShapeDtypeStruct((N, F), jnp.float32)] * 4,
    )(s1xp, s1hp, dis)

    s2xp, s2hp = _SPMV2(u1x, u1h, src, dst, zeros_nf)

    p = params
    wx = jnp.concatenate([p['Wx_i'], p['Wx_f'], p['Wx_c'], p['Wx_o']], axis=2)
    wh = jnp.concatenate([p['Wh_i'], p['Wh_f'], p['Wh_c'], p['Wh_o']], axis=2)
    bias = jnp.concatenate(
        [p['bx_' + g] + p['bh_' + g] + p['b_' + g][0]
         for g in ('i', 'f', 'c', 'o')]).reshape(1, GATES)
    wcs = jnp.concatenate([p['wc_i'], p['wc_f'], p['wc_o']], axis=0)

    hn, cn, sy, sy2 = pl.pallas_call(
        _gates_kernel,
        grid=(GRID,),
        in_specs=[_row_spec(F), _row_spec(F), _row_spec(F), _row_spec(F),
                  _part_spec(F), _row_spec(F), _part_spec(F), _row_spec(1),
                  _full_spec((3, F, GATES)), _full_spec((3, F, GATES)),
                  _full_spec((1, GATES)), _full_spec((3, F))],
        out_specs=[_row_spec(F), _row_spec(F),
                   pl.BlockSpec((1, F), lambda i: (0, 0)),
                   pl.BlockSpec((1, F), lambda i: (0, 0))],
        out_shape=[jax.ShapeDtypeStruct((N, F), jnp.float32),
                   jax.ShapeDtypeStruct((N, F), jnp.float32),
                   jax.ShapeDtypeStruct((1, F), jnp.float32),
                   jax.ShapeDtypeStruct((1, F), jnp.float32)],
    )(x, hidden_state, cell_state, tx1x, s2xp, tx1h, s2hp, dis,
      wx, wh, bias, wcs)

    m = sy / N
    m2 = sy2 / N
    gms = p['gn_mean_scale'][None, :]
    var = m2 - 2.0 * gms * m * m + gms * gms * m * m
    a = p['gn_weight'][None, :] * lax.rsqrt(var + 1e-5)
    shift = p['gn_bias'][None, :] - a * gms * m

    xp, yr = pl.pallas_call(
        _proj_kernel,
        grid=(GRID,),
        in_specs=[_row_spec(F), _full_spec((1, F)), _full_spec((1, F)),
                  _full_spec((F, F)), _full_spec((1, F)), _full_spec((F, 1))],
        out_specs=[_row_spec(F), _row_spec(1)],
        out_shape=[jax.ShapeDtypeStruct((N, F), jnp.float32),
                   jax.ShapeDtypeStruct((N, 1), jnp.float32)],
    )(hn, a, shift, p['W_proj'], p['b_proj'][None, :], p['W_r'])

    (sp,) = _SPMV1(xp, src, dst, zeros_nf)

    out = pl.pallas_call(
        _out_kernel,
        grid=(GRID,),
        in_specs=[_part_spec(F), _row_spec(1), _row_spec(1),
                  _full_spec((F, 1)), _full_spec((1, 1))],
        out_specs=_row_spec(1),
        out_shape=jax.ShapeDtypeStruct((N, 1), jnp.float32),
    )(sp, invc, yr, p['W_l'], p['b_l'][None, :])

    return out, hn, cn


# speedup vs baseline: 12.9593x; 1.1176x over previous
import functools

import jax
import jax.numpy as jnp
from jax import lax
from jax.experimental import pallas as pl
from jax.experimental.pallas import tpu as pltpu
from jax.experimental.pallas import tpu_sc as plsc

N = 10000
E = 320000
F = 128
GATES = 512

ROWS = 1000
GRID = N // ROWS

NC = 2
NS = 16
NW = NC * NS
EB = 128
NBLK = E // EB
WITER = (NBLK + NW - 1) // NW
DCH = 80
NCH = N // DCH
NP = 10240
DCH1 = 128
NCH1 = NP // DCH1


def _sc_mesh():
    return plsc.VectorSubcoreMesh(core_axis_name="c", subcore_axis_name="s")


def _zero_acc(sid, zeros_hbm, acc, dch, nch):
    citer = (nch + NS - 1) // NS

    def zbody(k, carry):
        ch = sid + k * NS

        @pl.when(ch < nch)
        def _():
            sl = pl.ds(ch * dch, dch)
            pltpu.sync_copy(zeros_hbm.at[sl], acc.at[sl])
        return carry

    lax.fori_loop(0, citer, zbody, 0)


def _dump_acc(sid, acc, bounce, out_view, dch, nch):
    citer = (nch + NS - 1) // NS

    def dbody(k, carry):
        ch = sid + k * NS

        @pl.when(ch < nch)
        def _():
            sl = pl.ds(ch * dch, dch)
            pltpu.sync_copy(acc.at[sl], bounce)
            pltpu.sync_copy(bounce, out_view.at[sl])
        return carry

    lax.fori_loop(0, citer, dbody, 0)


def _spmv_phase(wid, stride, sid, v_hbm, src_hbm, dst_hbm, zeros_hbm,
                out_view, bufs, bounce, acc):
    _zero_acc(sid, zeros_hbm, acc, DCH, NCH)
    plsc.subcore_barrier()

    def valid(i):
        return wid + i * stride < NBLK

    def base(i):
        return (wid + i * stride) * EB

    def idx_copies(i, buf):
        idxs, idxd = buf[0], buf[1]
        semi = buf[4]
        return (pltpu.make_async_copy(src_hbm.at[pl.ds(base(i), EB)],
                                      idxs, semi),
                pltpu.make_async_copy(dst_hbm.at[pl.ds(base(i), EB)],
                                      idxd, semi))

    def gather_copy(buf):
        return pltpu.make_async_copy(v_hbm.at[buf[0]], buf[2], buf[3])

    @pl.when(valid(0))
    def _():
        c0, c1 = idx_copies(0, bufs[0])
        c0.start()
        c1.start()
        c0.wait()
        c1.wait()
        gather_copy(bufs[0]).start()

    @pl.when(valid(1))
    def _():
        c0, c1 = idx_copies(1, bufs[1])
        c0.start()
        c1.start()

    def half(i, bp, bq):
        @pl.when(valid(i))
        def _():
            gather_copy(bp).wait()

            @pl.when(valid(i + 1))
            def _():
                c0, c1 = idx_copies(i + 1, bq)
                c0.wait()
                c1.wait()
                gather_copy(bq).start()

            pltpu.sync_copy(bp[2], acc.at[bp[1]], add=True)

            @pl.when(valid(i + 2))
            def _():
                c0, c1 = idx_copies(i + 2, bp)
                c0.start()
                c1.start()

    def ebody(j, carry):
        half(2 * j, bufs[0], bufs[1])
        half(2 * j + 1, bufs[1], bufs[0])
        return carry

    witer = (NBLK + stride - 1) // stride
    lax.fori_loop(0, (witer + 1) // 2, ebody, 0)
    plsc.subcore_barrier()
    _dump_acc(sid, acc, bounce, out_view, DCH, NCH)


_SPMV_SCRATCH = [
    pltpu.VMEM((EB,), jnp.int32),
    pltpu.VMEM((EB,), jnp.int32),
    pltpu.VMEM((EB, F), jnp.float32),
    pltpu.SemaphoreType.DMA,
    pltpu.SemaphoreType.DMA,
    pltpu.VMEM((EB,), jnp.int32),
    pltpu.VMEM((EB,), jnp.int32),
    pltpu.VMEM((EB, F), jnp.float32),
    pltpu.SemaphoreType.DMA,
    pltpu.SemaphoreType.DMA,
    pltpu.VMEM((DCH, F), jnp.float32),
    pltpu.VMEM_SHARED((N, F), jnp.float32),
]


@functools.partial(
    pl.kernel,
    mesh=_sc_mesh(),
    out_type=[jax.ShapeDtypeStruct((N, F), jnp.float32)] * 2,
    scratch_types=list(_SPMV_SCRATCH),
)
def _spmv_pair_sc(vx, vh, src_hbm, dst_hbm, zeros_hbm, outx, outh, *scr):
    bufs = (scr[0:5], scr[5:10])
    bounce, acc = scr[10], scr[11]
    cid = lax.axis_index("c")
    sid = lax.axis_index("s")

    @pl.when(cid == 0)
    def _():
        _spmv_phase(sid, NS, sid, vx, src_hbm, dst_hbm, zeros_hbm,
                    outx, bufs, bounce, acc)

    @pl.when(cid == 1)
    def _():
        _spmv_phase(sid, NS, sid, vh, src_hbm, dst_hbm, zeros_hbm,
                    outh, bufs, bounce, acc)


@functools.partial(
    pl.kernel,
    mesh=_sc_mesh(),
    out_type=jax.ShapeDtypeStruct((NC, N, F), jnp.float32),
    scratch_types=list(_SPMV_SCRATCH),
)
def _spmv_single_sc(v, src_hbm, dst_hbm, zeros_hbm, out, *scr):
    bufs = (scr[0:5], scr[5:10])
    bounce, acc = scr[10], scr[11]
    cid = lax.axis_index("c")
    sid = lax.axis_index("s")
    _spmv_phase(cid * NS + sid, NW, sid, v, src_hbm, dst_hbm, zeros_hbm,
                out.at[cid], bufs, bounce, acc)


@functools.partial(
    pl.kernel,
    mesh=_sc_mesh(),
    out_type=[jax.ShapeDtypeStruct((NC, NP), jnp.float32)] * 2,
    scratch_types=[
        pltpu.VMEM((EB,), jnp.int32),
        pltpu.VMEM((EB,), jnp.int32),
        pltpu.SemaphoreType.DMA,
        pltpu.VMEM((EB,), jnp.int32),
        pltpu.VMEM((EB,), jnp.int32),
        pltpu.SemaphoreType.DMA,
        pltpu.VMEM((EB,), jnp.float32),
        pltpu.VMEM((DCH1,), jnp.float32),
        pltpu.VMEM_SHARED((NP,), jnp.float32),
        pltpu.VMEM_SHARED((NP,), jnp.float32),
    ],
)
def _hist_sc(src_hbm, dst_hbm, zeros_hbm, deg_hbm, cnt_hbm,
             idxs0, idxd0, semi0, idxs1, idxd1, semi1,
             onesb, bounce, accd, accc):
    cid = lax.axis_index("c")
    sid = lax.axis_index("s")
    wid = cid * NS + sid
    bufs = ((idxs0, idxd0, semi0), (idxs1, idxd1, semi1))

    def obody(j, carry):
        onesb[pl.ds(j * 16, 16)] = jnp.ones((16,), jnp.float32)
        return carry

    lax.fori_loop(0, EB // 16, obody, 0)
    _zero_acc(sid, zeros_hbm, accd, DCH1, NCH1)
    _zero_acc(sid, zeros_hbm, accc, DCH1, NCH1)
    plsc.subcore_barrier()

    def valid(i):
        return wid + i * NW < NBLK

    def idx_copies(i, buf):
        base = (wid + i * NW) * EB
        return (pltpu.make_async_copy(src_hbm.at[pl.ds(base, EB)],
                                      buf[0], buf[2]),
                pltpu.make_async_copy(dst_hbm.at[pl.ds(base, EB)],
                                      buf[1], buf[2]))

    @pl.when(valid(0))
    def _():
        c0, c1 = idx_copies(0, bufs[0])
        c0.start()
        c1.start()

    def half(i, bp, bq):
        @pl.when(valid(i))
        def _():
            c0, c1 = idx_copies(i, bp)
            c0.wait()
            c1.wait()

            @pl.when(valid(i + 1))
            def _():
                n0, n1 = idx_copies(i + 1, bq)
                n0.start()
                n1.start()

            pltpu.sync_copy(onesb, accd.at[bp[0]], add=True)
            pltpu.sync_copy(onesb, accc.at[bp[1]], add=True)

    def ebody(j, carry):
        half(2 * j, bufs[0], bufs[1])
        half(2 * j + 1, bufs[1], bufs[0])
        return carry

    lax.fori_loop(0, (WITER + 1) // 2, ebody, 0)
    plsc.subcore_barrier()
    _dump_acc(sid, accd, bounce, deg_hbm.at[cid], DCH1, NCH1)
    _dump_acc(sid, accc, bounce, cnt_hbm.at[cid], DCH1, NCH1)


def _prep_kernel(degp_ref, cntp_ref, x_ref, h_ref,
                 dis_ref, invc_ref, u0x_ref, u0h_ref):
    deg = degp_ref[0, :, 0] + degp_ref[1, :, 0]
    cnt = cntp_ref[0, :, 0] + cntp_ref[1, :, 0]
    dis = jnp.where(deg > 0, lax.rsqrt(jnp.where(deg > 0, deg, 1.0)), 0.0)
    dis_ref[...] = dis[:, None]
    invc_ref[...] = (1.0 / jnp.maximum(cnt, 1.0))[:, None]
    u0x_ref[...] = dis[:, None] * x_ref[...]
    u0h_ref[...] = dis[:, None] * h_ref[...]


def _mid_kernel(s1x_ref, s1h_ref, dis_ref,
                tx1x_ref, u1x_ref, tx1h_ref, u1h_ref):
    dis = dis_ref[...]
    tx1x = -dis * s1x_ref[...]
    tx1h = -dis * s1h_ref[...]
    tx1x_ref[...] = tx1x
    tx1h_ref[...] = tx1h
    u1x_ref[...] = dis * tx1x
    u1h_ref[...] = dis * tx1h


def _gates_kernel(x_ref, h_ref, c_ref, tx1x_ref, s2x_ref, tx1h_ref,
                  s2h_ref, dis_ref, wx_ref, wh_ref, bias_ref, wcs_ref,
                  hn_ref, cn_ref, sy_ref, sy2_ref):
    i = pl.program_id(0)
    dis = dis_ref[...]
    x = x_ref[...]
    h = h_ref[...]
    c = c_ref[...]
    tx2x = -2.0 * dis * s2x_ref[...] - x
    tx2h = -2.0 * dis * s2h_ref[...] - h

    def mm(a, w):
        return jnp.dot(a, w, preferred_element_type=jnp.float32)

    z = (mm(x, wx_ref[0]) + mm(tx1x_ref[...], wx_ref[1]) + mm(tx2x, wx_ref[2])
         + mm(h, wh_ref[0]) + mm(tx1h_ref[...], wh_ref[1]) + mm(tx2h, wh_ref[2])
         + bias_ref[...])
    gi = jax.nn.sigmoid(z[:, 0:F] + wcs_ref[0:1, :] * c)
    gf = jax.nn.sigmoid(z[:, F:2 * F] + wcs_ref[1:2, :] * c)
    gt = jnp.tanh(z[:, 2 * F:3 * F])
    cn = gf * c + gi * gt
    go = jax.nn.sigmoid(z[:, 3 * F:4 * F] + wcs_ref[2:3, :] * cn)
    hn = go * jnp.tanh(cn)
    hn_ref[...] = hn
    cn_ref[...] = cn
    y = jnp.maximum(hn, 0.0)

    @pl.when(i == 0)
    def _():
        sy_ref[...] = jnp.zeros_like(sy_ref)
        sy2_ref[...] = jnp.zeros_like(sy2_ref)

    sy_ref[...] += jnp.sum(y, axis=0, keepdims=True)
    sy2_ref[...] += jnp.sum(y * y, axis=0, keepdims=True)


def _proj_kernel(hn_ref, a_ref, shift_ref, wproj_ref, bproj_ref, wr_ref,
                 xp_ref, yr_ref):
    y = a_ref[...] * jnp.maximum(hn_ref[...], 0.0) + shift_ref[...]
    xp = jnp.dot(y, wproj_ref[...], preferred_element_type=jnp.float32)
    xp_ref[...] = jnp.maximum(xp + bproj_ref[...], 0.0)
    yr_ref[...] = jnp.dot(y, wr_ref[...], preferred_element_type=jnp.float32)


def _out_kernel(sp_ref, invc_ref, yr_ref, wl_ref, bl_ref, out_ref):
    mean_nb = (sp_ref[0] + sp_ref[1]) * invc_ref[...]
    out_ref[...] = (jnp.dot(mean_nb, wl_ref[...],
                            preferred_element_type=jnp.float32)
                    + bl_ref[...] + yr_ref[...])


def _row_spec(width):
    return pl.BlockSpec((ROWS, width), lambda i: (i, 0))


def _part_spec(width):
    return pl.BlockSpec((2, ROWS, width), lambda i: (0, i, 0))


def _full_spec(shape):
    return pl.BlockSpec(shape, lambda i: tuple(0 for _ in shape))


def kernel(x, edge_index, hidden_state, cell_state, edge_weights, params):
    src, dst = edge_index[0], edge_index[1]
    zeros_nf = jnp.zeros((N, F), jnp.float32)
    zeros_np = jnp.zeros((NP,), jnp.float32)

    degp2, cntp2 = _hist_sc(src, dst, zeros_np)
    degp = degp2[:, :N, None]
    cntp = cntp2[:, :N, None]

    dis, invc, u0x, u0h = pl.pallas_call(
        _prep_kernel,
        grid=(GRID,),
        in_specs=[pl.BlockSpec((2, ROWS, 1), lambda i: (0, i, 0)),
                  pl.BlockSpec((2, ROWS, 1), lambda i: (0, i, 0)),
                  _row_spec(F), _row_spec(F)],
        out_specs=[_row_spec(1), _row_spec(1), _row_spec(F), _row_spec(F)],
        out_shape=[jax.ShapeDtypeStruct((N, 1), jnp.float32),
                   jax.ShapeDtypeStruct((N, 1), jnp.float32),
                   jax.ShapeDtypeStruct((N, F), jnp.float32),
                   jax.ShapeDtypeStruct((N, F), jnp.float32)],
    )(degp, cntp, x, hidden_state)

    s1x, s1h = _spmv_pair_sc(u0x, u0h, src, dst, zeros_nf)

    tx1x, u1x, tx1h, u1h = pl.pallas_call(
        _mid_kernel,
        grid=(GRID,),
        in_specs=[_row_spec(F), _row_spec(F), _row_spec(1)],
        out_specs=[_row_spec(F)] * 4,
        out_shape=[jax.ShapeDtypeStruct((N, F), jnp.float32)] * 4,
    )(s1x, s1h, dis)

    s2x, s2h = _spmv_pair_sc(u1x, u1h, src, dst, zeros_nf)

    p = params
    wx = jnp.concatenate([p['Wx_i'], p['Wx_f'], p['Wx_c'], p['Wx_o']], axis=2)
    wh = jnp.concatenate([p['Wh_i'], p['Wh_f'], p['Wh_c'], p['Wh_o']], axis=2)
    bias = jnp.concatenate(
        [p['bx_' + g] + p['bh_' + g] + p['b_' + g][0]
         for g in ('i', 'f', 'c', 'o')]).reshape(1, GATES)
    wcs = jnp.concatenate([p['wc_i'], p['wc_f'], p['wc_o']], axis=0)

    hn, cn, sy, sy2 = pl.pallas_call(
        _gates_kernel,
        grid=(GRID,),
        in_specs=[_row_spec(F), _row_spec(F), _row_spec(F), _row_spec(F),
                  _row_spec(F), _row_spec(F), _row_spec(F), _row_spec(1),
                  _full_spec((3, F, GATES)), _full_spec((3, F, GATES)),
                  _full_spec((1, GATES)), _full_spec((3, F))],
        out_specs=[_row_spec(F), _row_spec(F),
                   pl.BlockSpec((1, F), lambda i: (0, 0)),
                   pl.BlockSpec((1, F), lambda i: (0, 0))],
        out_shape=[jax.ShapeDtypeStruct((N, F), jnp.float32),
                   jax.ShapeDtypeStruct((N, F), jnp.float32),
                   jax.ShapeDtypeStruct((1, F), jnp.float32),
                   jax.ShapeDtypeStruct((1, F), jnp.float32)],
    )(x, hidden_state, cell_state, tx1x, s2x, tx1h, s2h, dis,
      wx, wh, bias, wcs)

    m = sy / N
    m2 = sy2 / N
    gms = p['gn_mean_scale'][None, :]
    var = m2 - 2.0 * gms * m * m + gms * gms * m * m
    a = p['gn_weight'][None, :] * lax.rsqrt(var + 1e-5)
    shift = p['gn_bias'][None, :] - a * gms * m

    xp, yr = pl.pallas_call(
        _proj_kernel,
        grid=(GRID,),
        in_specs=[_row_spec(F), _full_spec((1, F)), _full_spec((1, F)),
                  _full_spec((F, F)), _full_spec((1, F)), _full_spec((F, 1))],
        out_specs=[_row_spec(F), _row_spec(1)],
        out_shape=[jax.ShapeDtypeStruct((N, F), jnp.float32),
                   jax.ShapeDtypeStruct((N, 1), jnp.float32)],
    )(hn, a, shift, p['W_proj'], p['b_proj'][None, :], p['W_r'])

    sp = _spmv_single_sc(xp, src, dst, zeros_nf)

    out = pl.pallas_call(
        _out_kernel,
        grid=(GRID,),
        in_specs=[_part_spec(F), _row_spec(1), _row_spec(1),
                  _full_spec((F, 1)), _full_spec((1, 1))],
        out_specs=_row_spec(1),
        out_shape=jax.ShapeDtypeStruct((N, 1), jnp.float32),
    )(sp, invc, yr, p['W_l'], p['b_l'][None, :])

    return out, hn, cn


# speedup vs baseline: 12.9718x; 1.0010x over previous
import functools

import jax
import jax.numpy as jnp
from jax import lax
from jax.experimental import pallas as pl
from jax.experimental.pallas import tpu as pltpu
from jax.experimental.pallas import tpu_sc as plsc

N = 10000
E = 320000
F = 128
GATES = 512

ROWS = 1000
GRID = N // ROWS

NC = 2
NS = 16
NW = NC * NS
EB = 128
NBLK = E // EB
WITER = (NBLK + NW - 1) // NW
DCH = 80
NCH = N // DCH
NP = 10240
DCH1 = 128
NCH1 = NP // DCH1


def _sc_mesh():
    return plsc.VectorSubcoreMesh(core_axis_name="c", subcore_axis_name="s")


def _zero_acc(sid, zeros_hbm, acc, dch, nch):
    citer = (nch + NS - 1) // NS

    def zbody(k, carry):
        ch = sid + k * NS

        @pl.when(ch < nch)
        def _():
            sl = pl.ds(ch * dch, dch)
            pltpu.sync_copy(zeros_hbm.at[sl], acc.at[sl])
        return carry

    lax.fori_loop(0, citer, zbody, 0)


def _dump_acc(sid, acc, bounce, out_view, dch, nch):
    citer = (nch + NS - 1) // NS

    def dbody(k, carry):
        ch = sid + k * NS

        @pl.when(ch < nch)
        def _():
            sl = pl.ds(ch * dch, dch)
            pltpu.sync_copy(acc.at[sl], bounce)
            pltpu.sync_copy(bounce, out_view.at[sl])
        return carry

    lax.fori_loop(0, citer, dbody, 0)


def _spmv_phase(wid, stride, sid, v_hbm, src_hbm, dst_hbm, zeros_hbm,
                out_view, bufs, bounce, acc):
    ibufs, rbufs = bufs
    _zero_acc(sid, zeros_hbm, acc, DCH, NCH)
    plsc.subcore_barrier()

    def valid(i):
        return wid + i * stride < NBLK

    def base(i):
        return (wid + i * stride) * EB

    def idx_copies(i, k):
        ib = ibufs[k]
        return (pltpu.make_async_copy(src_hbm.at[pl.ds(base(i), EB)],
                                      ib[0], ib[2]),
                pltpu.make_async_copy(dst_hbm.at[pl.ds(base(i), EB)],
                                      ib[1], ib[2]))

    def gather_copy(k, p):
        return pltpu.make_async_copy(v_hbm.at[ibufs[k][0]],
                                     rbufs[p][0], rbufs[p][1])

    def scatter_copy(k, p):
        return pltpu.make_async_copy(rbufs[p][0], acc.at[ibufs[k][1]],
                                     rbufs[p][2])

    @pl.when(valid(0))
    def _():
        c0, c1 = idx_copies(0, 0)
        c0.start()
        c1.start()
        c0.wait()
        c1.wait()
        gather_copy(0, 0).start()

    @pl.when(valid(1))
    def _():
        c0, c1 = idx_copies(1, 1)
        c0.start()
        c1.start()

    def step(i, k, p):
        kn = (k + 1) % 3
        kp = (k + 2) % 3
        q = 1 - p

        @pl.when(valid(i))
        def _():
            gather_copy(k, p).wait()

        @pl.when((i >= 1) & valid(i - 1))
        def _():
            scatter_copy(kp, q).wait()

        @pl.when(valid(i + 1))
        def _():
            c0, c1 = idx_copies(i + 1, kn)
            c0.wait()
            c1.wait()
            gather_copy(kn, q).start()

        @pl.when(valid(i))
        def _():
            pltpu.async_copy(rbufs[p][0], acc.at[ibufs[k][1]],
                             rbufs[p][2], add=True)

        @pl.when(valid(i + 2))
        def _():
            c0, c1 = idx_copies(i + 2, kp)
            c0.start()
            c1.start()

    def ebody(j, carry):
        for u in range(6):
            step(6 * j + u, u % 3, u % 2)
        return carry

    witer = (NBLK + stride - 1) // stride
    nsteps = ((witer + 1) + 5) // 6
    lax.fori_loop(0, nsteps, ebody, 0)
    plsc.subcore_barrier()
    _dump_acc(sid, acc, bounce, out_view, DCH, NCH)


_SPMV_SCRATCH = [
    pltpu.VMEM((EB,), jnp.int32), pltpu.VMEM((EB,), jnp.int32),
    pltpu.SemaphoreType.DMA,
    pltpu.VMEM((EB,), jnp.int32), pltpu.VMEM((EB,), jnp.int32),
    pltpu.SemaphoreType.DMA,
    pltpu.VMEM((EB,), jnp.int32), pltpu.VMEM((EB,), jnp.int32),
    pltpu.SemaphoreType.DMA,
    pltpu.VMEM((EB, F), jnp.float32),
    pltpu.SemaphoreType.DMA, pltpu.SemaphoreType.DMA,
    pltpu.VMEM((EB, F), jnp.float32),
    pltpu.SemaphoreType.DMA, pltpu.SemaphoreType.DMA,
    pltpu.VMEM((DCH, F), jnp.float32),
    pltpu.VMEM_SHARED((N, F), jnp.float32),
]


@functools.partial(
    pl.kernel,
    mesh=_sc_mesh(),
    out_type=[jax.ShapeDtypeStruct((N, F), jnp.float32)] * 2,
    scratch_types=list(_SPMV_SCRATCH),
)
def _spmv_pair_sc(vx, vh, src_hbm, dst_hbm, zeros_hbm, outx, outh, *scr):
    bufs = ((scr[0:3], scr[3:6], scr[6:9]),
            (scr[9:12], scr[12:15]))
    bounce, acc = scr[15], scr[16]
    cid = lax.axis_index("c")
    sid = lax.axis_index("s")

    @pl.when(cid == 0)
    def _():
        _spmv_phase(sid, NS, sid, vx, src_hbm, dst_hbm, zeros_hbm,
                    outx, bufs, bounce, acc)

    @pl.when(cid == 1)
    def _():
        _spmv_phase(sid, NS, sid, vh, src_hbm, dst_hbm, zeros_hbm,
                    outh, bufs, bounce, acc)


@functools.partial(
    pl.kernel,
    mesh=_sc_mesh(),
    out_type=jax.ShapeDtypeStruct((NC, N, F), jnp.float32),
    scratch_types=list(_SPMV_SCRATCH),
)
def _spmv_single_sc(v, src_hbm, dst_hbm, zeros_hbm, out, *scr):
    bufs = ((scr[0:3], scr[3:6], scr[6:9]),
            (scr[9:12], scr[12:15]))
    bounce, acc = scr[15], scr[16]
    cid = lax.axis_index("c")
    sid = lax.axis_index("s")
    _spmv_phase(cid * NS + sid, NW, sid, v, src_hbm, dst_hbm, zeros_hbm,
                out.at[cid], bufs, bounce, acc)


@functools.partial(
    pl.kernel,
    mesh=_sc_mesh(),
    out_type=[jax.ShapeDtypeStruct((NC, NP), jnp.float32)] * 2,
    scratch_types=[
        pltpu.VMEM((EB,), jnp.int32),
        pltpu.VMEM((EB,), jnp.int32),
        pltpu.SemaphoreType.DMA,
        pltpu.VMEM((EB,), jnp.int32),
        pltpu.VMEM((EB,), jnp.int32),
        pltpu.SemaphoreType.DMA,
        pltpu.VMEM((EB,), jnp.float32),
        pltpu.VMEM((DCH1,), jnp.float32),
        pltpu.VMEM_SHARED((NP,), jnp.float32),
        pltpu.VMEM_SHARED((NP,), jnp.float32),
    ],
)
def _hist_sc(src_hbm, dst_hbm, zeros_hbm, deg_hbm, cnt_hbm,
             idxs0, idxd0, semi0, idxs1, idxd1, semi1,
             onesb, bounce, accd, accc):
    cid = lax.axis_index("c")
    sid = lax.axis_index("s")
    wid = cid * NS + sid
    bufs = ((idxs0, idxd0, semi0), (idxs1, idxd1, semi1))

    def obody(j, carry):
        onesb[pl.ds(j * 16, 16)] = jnp.ones((16,), jnp.float32)
        return carry

    lax.fori_loop(0, EB // 16, obody, 0)
    _zero_acc(sid, zeros_hbm, accd, DCH1, NCH1)
    _zero_acc(sid, zeros_hbm, accc, DCH1, NCH1)
    plsc.subcore_barrier()

    def valid(i):
        return wid + i * NW < NBLK

    def idx_copies(i, buf):
        base = (wid + i * NW) * EB
        return (pltpu.make_async_copy(src_hbm.at[pl.ds(base, EB)],
                                      buf[0], buf[2]),
                pltpu.make_async_copy(dst_hbm.at[pl.ds(base, EB)],
                                      buf[1], buf[2]))

    @pl.when(valid(0))
    def _():
        c0, c1 = idx_copies(0, bufs[0])
        c0.start()
        c1.start()

    def half(i, bp, bq):
        @pl.when(valid(i))
        def _():
            c0, c1 = idx_copies(i, bp)
            c0.wait()
            c1.wait()

            @pl.when(valid(i + 1))
            def _():
                n0, n1 = idx_copies(i + 1, bq)
                n0.start()
                n1.start()

            pltpu.sync_copy(onesb, accd.at[bp[0]], add=True)
            pltpu.sync_copy(onesb, accc.at[bp[1]], add=True)

    def ebody(j, carry):
        half(2 * j, bufs[0], bufs[1])
        half(2 * j + 1, bufs[1], bufs[0])
        return carry

    lax.fori_loop(0, (WITER + 1) // 2, ebody, 0)
    plsc.subcore_barrier()
    _dump_acc(sid, accd, bounce, deg_hbm.at[cid], DCH1, NCH1)
    _dump_acc(sid, accc, bounce, cnt_hbm.at[cid], DCH1, NCH1)


def _prep_kernel(degp_ref, cntp_ref, x_ref, h_ref,
                 dis_ref, invc_ref, u0x_ref, u0h_ref):
    deg = degp_ref[0, :, 0] + degp_ref[1, :, 0]
    cnt = cntp_ref[0, :, 0] + cntp_ref[1, :, 0]
    dis = jnp.where(deg > 0, lax.rsqrt(jnp.where(deg > 0, deg, 1.0)), 0.0)
    dis_ref[...] = dis[:, None]
    invc_ref[...] = (1.0 / jnp.maximum(cnt, 1.0))[:, None]
    u0x_ref[...] = dis[:, None] * x_ref[...]
    u0h_ref[...] = dis[:, None] * h_ref[...]


def _mid_kernel(s1x_ref, s1h_ref, dis_ref,
                tx1x_ref, u1x_ref, tx1h_ref, u1h_ref):
    dis = dis_ref[...]
    tx1x = -dis * s1x_ref[...]
    tx1h = -dis * s1h_ref[...]
    tx1x_ref[...] = tx1x
    tx1h_ref[...] = tx1h
    u1x_ref[...] = dis * tx1x
    u1h_ref[...] = dis * tx1h


def _gates_kernel(x_ref, h_ref, c_ref, tx1x_ref, s2x_ref, tx1h_ref,
                  s2h_ref, dis_ref, wx_ref, wh_ref, bias_ref, wcs_ref,
                  hn_ref, cn_ref, sy_ref, sy2_ref):
    i = pl.program_id(0)
    dis = dis_ref[...]
    x = x_ref[...]
    h = h_ref[...]
    c = c_ref[...]
    tx2x = -2.0 * dis * s2x_ref[...] - x
    tx2h = -2.0 * dis * s2h_ref[...] - h

    def mm(a, w):
        return jnp.dot(a, w, preferred_element_type=jnp.float32)

    z = (mm(x, wx_ref[0]) + mm(tx1x_ref[...], wx_ref[1]) + mm(tx2x, wx_ref[2])
         + mm(h, wh_ref[0]) + mm(tx1h_ref[...], wh_ref[1]) + mm(tx2h, wh_ref[2])
         + bias_ref[...])
    gi = jax.nn.sigmoid(z[:, 0:F] + wcs_ref[0:1, :] * c)
    gf = jax.nn.sigmoid(z[:, F:2 * F] + wcs_ref[1:2, :] * c)
    gt = jnp.tanh(z[:, 2 * F:3 * F])
    cn = gf * c + gi * gt
    go = jax.nn.sigmoid(z[:, 3 * F:4 * F] + wcs_ref[2:3, :] * cn)
    hn = go * jnp.tanh(cn)
    hn_ref[...] = hn
    cn_ref[...] = cn
    y = jnp.maximum(hn, 0.0)

    @pl.when(i == 0)
    def _():
        sy_ref[...] = jnp.zeros_like(sy_ref)
        sy2_ref[...] = jnp.zeros_like(sy2_ref)

    sy_ref[...] += jnp.sum(y, axis=0, keepdims=True)
    sy2_ref[...] += jnp.sum(y * y, axis=0, keepdims=True)


def _proj_kernel(hn_ref, a_ref, shift_ref, wproj_ref, bproj_ref, wr_ref,
                 xp_ref, yr_ref):
    y = a_ref[...] * jnp.maximum(hn_ref[...], 0.0) + shift_ref[...]
    xp = jnp.dot(y, wproj_ref[...], preferred_element_type=jnp.float32)
    xp_ref[...] = jnp.maximum(xp + bproj_ref[...], 0.0)
    yr_ref[...] = jnp.dot(y, wr_ref[...], preferred_element_type=jnp.float32)


def _out_kernel(sp_ref, invc_ref, yr_ref, wl_ref, bl_ref, out_ref):
    mean_nb = (sp_ref[0] + sp_ref[1]) * invc_ref[...]
    out_ref[...] = (jnp.dot(mean_nb, wl_ref[...],
                            preferred_element_type=jnp.float32)
                    + bl_ref[...] + yr_ref[...])


def _row_spec(width):
    return pl.BlockSpec((ROWS, width), lambda i: (i, 0))


def _part_spec(width):
    return pl.BlockSpec((2, ROWS, width), lambda i: (0, i, 0))


def _full_spec(shape):
    return pl.BlockSpec(shape, lambda i: tuple(0 for _ in shape))


def kernel(x, edge_index, hidden_state, cell_state, edge_weights, params):
    src, dst = edge_index[0], edge_index[1]
    zeros_nf = jnp.zeros((N, F), jnp.float32)
    zeros_np = jnp.zeros((NP,), jnp.float32)

    degp2, cntp2 = _hist_sc(src, dst, zeros_np)
    degp = degp2[:, :N, None]
    cntp = cntp2[:, :N, None]

    dis, invc, u0x, u0h = pl.pallas_call(
        _prep_kernel,
        grid=(GRID,),
        in_specs=[pl.BlockSpec((2, ROWS, 1), lambda i: (0, i, 0)),
                  pl.BlockSpec((2, ROWS, 1), lambda i: (0, i, 0)),
                  _row_spec(F), _row_spec(F)],
        out_specs=[_row_spec(1), _row_spec(1), _row_spec(F), _row_spec(F)],
        out_shape=[jax.ShapeDtypeStruct((N, 1), jnp.float32),
                   jax.ShapeDtypeStruct((N, 1), jnp.float32),
                   jax.ShapeDtypeStruct((N, F), jnp.float32),
                   jax.ShapeDtypeStruct((N, F), jnp.float32)],
    )(degp, cntp, x, hidden_state)

    s1x, s1h = _spmv_pair_sc(u0x, u0h, src, dst, zeros_nf)

    tx1x, u1x, tx1h, u1h = pl.pallas_call(
        _mid_kernel,
        grid=(GRID,),
        in_specs=[_row_spec(F), _row_spec(F), _row_spec(1)],
        out_specs=[_row_spec(F)] * 4,
        out_shape=[jax.ShapeDtypeStruct((N, F), jnp.float32)] * 4,
    )(s1x, s1h, dis)

    s2x, s2h = _spmv_pair_sc(u1x, u1h, src, dst, zeros_nf)

    p = params
    wx = jnp.concatenate([p['Wx_i'], p['Wx_f'], p['Wx_c'], p['Wx_o']], axis=2)
    wh = jnp.concatenate([p['Wh_i'], p['Wh_f'], p['Wh_c'], p['Wh_o']], axis=2)
    bias = jnp.concatenate(
        [p['bx_' + g] + p['bh_' + g] + p['b_' + g][0]
         for g in ('i', 'f', 'c', 'o')]).reshape(1, GATES)
    wcs = jnp.concatenate([p['wc_i'], p['wc_f'], p['wc_o']], axis=0)

    hn, cn, sy, sy2 = pl.pallas_call(
        _gates_kernel,
        grid=(GRID,),
        in_specs=[_row_spec(F), _row_spec(F), _row_spec(F), _row_spec(F),
                  _row_spec(F), _row_spec(F), _row_spec(F), _row_spec(1),
                  _full_spec((3, F, GATES)), _full_spec((3, F, GATES)),
                  _full_spec((1, GATES)), _full_spec((3, F))],
        out_specs=[_row_spec(F), _row_spec(F),
                   pl.BlockSpec((1, F), lambda i: (0, 0)),
                   pl.BlockSpec((1, F), lambda i: (0, 0))],
        out_shape=[jax.ShapeDtypeStruct((N, F), jnp.float32),
                   jax.ShapeDtypeStruct((N, F), jnp.float32),
                   jax.ShapeDtypeStruct((1, F), jnp.float32),
                   jax.ShapeDtypeStruct((1, F), jnp.float32)],
    )(x, hidden_state, cell_state, tx1x, s2x, tx1h, s2h, dis,
      wx, wh, bias, wcs)

    m = sy / N
    m2 = sy2 / N
    gms = p['gn_mean_scale'][None, :]
    var = m2 - 2.0 * gms * m * m + gms * gms * m * m
    a = p['gn_weight'][None, :] * lax.rsqrt(var + 1e-5)
    shift = p['gn_bias'][None, :] - a * gms * m

    xp, yr = pl.pallas_call(
        _proj_kernel,
        grid=(GRID,),
        in_specs=[_row_spec(F), _full_spec((1, F)), _full_spec((1, F)),
                  _full_spec((F, F)), _full_spec((1, F)), _full_spec((F, 1))],
        out_specs=[_row_spec(F), _row_spec(1)],
        out_shape=[jax.ShapeDtypeStruct((N, F), jnp.float32),
                   jax.ShapeDtypeStruct((N, 1), jnp.float32)],
    )(hn, a, shift, p['W_proj'], p['b_proj'][None, :], p['W_r'])

    sp = _spmv_single_sc(xp, src, dst, zeros_nf)

    out = pl.pallas_call(
        _out_kernel,
        grid=(GRID,),
        in_specs=[_part_spec(F), _row_spec(1), _row_spec(1),
                  _full_spec((F, 1)), _full_spec((1, 1))],
        out_specs=_row_spec(1),
        out_shape=jax.ShapeDtypeStruct((N, 1), jnp.float32),
    )(sp, invc, yr, p['W_l'], p['b_l'][None, :])

    return out, hn, cn


# speedup vs baseline: 15.0598x; 1.1610x over previous
import functools

import jax
import jax.numpy as jnp
from jax import lax
from jax.experimental import pallas as pl
from jax.experimental.pallas import tpu as pltpu
from jax.experimental.pallas import tpu_sc as plsc

N = 10000
E = 320000
F = 128
GATES = 512

ROWS = 1000
GRID = N // ROWS

NC = 2
NS = 16
NW = NC * NS
EB = 128
NBLK = E // EB
WITER = (NBLK + NW - 1) // NW
DCH = 80
NCH = N // DCH
NP = 10240
DCH1 = 128
NCH1 = NP // DCH1


def _sc_mesh():
    return plsc.VectorSubcoreMesh(core_axis_name="c", subcore_axis_name="s")


def _zero_acc(sid, zeros_hbm, acc, dch, nch):
    citer = (nch + NS - 1) // NS

    def zbody(k, carry):
        ch = sid + k * NS

        @pl.when(ch < nch)
        def _():
            sl = pl.ds(ch * dch, dch)
            pltpu.sync_copy(zeros_hbm.at[sl], acc.at[sl])
        return carry

    lax.fori_loop(0, citer, zbody, 0)


def _dump_acc(sid, acc, bounce, out_view, dch, nch):
    citer = (nch + NS - 1) // NS

    def dbody(k, carry):
        ch = sid + k * NS

        @pl.when(ch < nch)
        def _():
            sl = pl.ds(ch * dch, dch)
            pltpu.sync_copy(acc.at[sl], bounce)
            pltpu.sync_copy(bounce, out_view.at[sl])
        return carry

    lax.fori_loop(0, citer, dbody, 0)


def _spmv_phase(wid, stride, sid, v_hbm, src_hbm, dst_hbm, zeros_hbm,
                out_view, bufs, bounce, acc):
    ibufs, rbufs = bufs
    _zero_acc(sid, zeros_hbm, acc, DCH, NCH)
    plsc.subcore_barrier()

    def valid(i):
        return wid + i * stride < NBLK

    def base(i):
        return (wid + i * stride) * EB

    def idx_copies(i, k):
        ib = ibufs[k]
        return (pltpu.make_async_copy(src_hbm.at[pl.ds(base(i), EB)],
                                      ib[0], ib[2]),
                pltpu.make_async_copy(dst_hbm.at[pl.ds(base(i), EB)],
                                      ib[1], ib[2]))

    def gather_copy(k, p):
        return pltpu.make_async_copy(v_hbm.at[ibufs[k][0]],
                                     rbufs[p][0], rbufs[p][1])

    def scatter_copy(k, p):
        return pltpu.make_async_copy(rbufs[p][0], acc.at[ibufs[k][1]],
                                     rbufs[p][2])

    @pl.when(valid(0))
    def _():
        c0, c1 = idx_copies(0, 0)
        c0.start()
        c1.start()
        c0.wait()
        c1.wait()
        gather_copy(0, 0).start()

    @pl.when(valid(1))
    def _():
        c0, c1 = idx_copies(1, 1)
        c0.start()
        c1.start()

    def step(i, k, p):
        kn = (k + 1) % 3
        kp = (k + 2) % 3
        q = 1 - p

        @pl.when((i >= 1) & valid(i - 1))
        def _():
            scatter_copy(kp, q).wait()

        @pl.when(valid(i + 1))
        def _():
            c0, c1 = idx_copies(i + 1, kn)
            c0.wait()
            c1.wait()
            gather_copy(kn, q).start()

        @pl.when(valid(i))
        def _():
            gather_copy(k, p).wait()
            pltpu.async_copy(rbufs[p][0], acc.at[ibufs[k][1]],
                             rbufs[p][2], add=True)

        @pl.when(valid(i + 2))
        def _():
            c0, c1 = idx_copies(i + 2, kp)
            c0.start()
            c1.start()

    def ebody(j, carry):
        for u in range(6):
            step(6 * j + u, u % 3, u % 2)
        return carry

    witer = (NBLK + stride - 1) // stride
    nsteps = ((witer + 1) + 5) // 6
    lax.fori_loop(0, nsteps, ebody, 0)
    plsc.subcore_barrier()
    _dump_acc(sid, acc, bounce, out_view, DCH, NCH)


_SPMV_SCRATCH = [
    pltpu.VMEM((EB,), jnp.int32), pltpu.VMEM((EB,), jnp.int32),
    pltpu.SemaphoreType.DMA,
    pltpu.VMEM((EB,), jnp.int32), pltpu.VMEM((EB,), jnp.int32),
    pltpu.SemaphoreType.DMA,
    pltpu.VMEM((EB,), jnp.int32), pltpu.VMEM((EB,), jnp.int32),
    pltpu.SemaphoreType.DMA,
    pltpu.VMEM((EB, F), jnp.float32),
    pltpu.SemaphoreType.DMA, pltpu.SemaphoreType.DMA,
    pltpu.VMEM((EB, F), jnp.float32),
    pltpu.SemaphoreType.DMA, pltpu.SemaphoreType.DMA,
    pltpu.VMEM((DCH, F), jnp.float32),
    pltpu.VMEM_SHARED((N, F), jnp.float32),
]


@functools.partial(
    pl.kernel,
    mesh=_sc_mesh(),
    out_type=[jax.ShapeDtypeStruct((N, F), jnp.float32)] * 2,
    scratch_types=list(_SPMV_SCRATCH),
)
def _spmv_pair_sc(vx, vh, src_hbm, dst_hbm, zeros_hbm, outx, outh, *scr):
    bufs = ((scr[0:3], scr[3:6], scr[6:9]),
            (scr[9:12], scr[12:15]))
    bounce, acc = scr[15], scr[16]
    cid = lax.axis_index("c")
    sid = lax.axis_index("s")

    @pl.when(cid == 0)
    def _():
        _spmv_phase(sid, NS, sid, vx, src_hbm, dst_hbm, zeros_hbm,
                    outx, bufs, bounce, acc)

    @pl.when(cid == 1)
    def _():
        _spmv_phase(sid, NS, sid, vh, src_hbm, dst_hbm, zeros_hbm,
                    outh, bufs, bounce, acc)


@functools.partial(
    pl.kernel,
    mesh=_sc_mesh(),
    out_type=jax.ShapeDtypeStruct((NC, N, F), jnp.float32),
    scratch_types=list(_SPMV_SCRATCH),
)
def _spmv_single_sc(v, src_hbm, dst_hbm, zeros_hbm, out, *scr):
    bufs = ((scr[0:3], scr[3:6], scr[6:9]),
            (scr[9:12], scr[12:15]))
    bounce, acc = scr[15], scr[16]
    cid = lax.axis_index("c")
    sid = lax.axis_index("s")
    _spmv_phase(cid * NS + sid, NW, sid, v, src_hbm, dst_hbm, zeros_hbm,
                out.at[cid], bufs, bounce, acc)


@functools.partial(
    pl.kernel,
    mesh=_sc_mesh(),
    out_type=[jax.ShapeDtypeStruct((NC, NP), jnp.float32)] * 2,
    scratch_types=[
        pltpu.VMEM((EB,), jnp.int32),
        pltpu.VMEM((EB,), jnp.int32),
        pltpu.SemaphoreType.DMA,
        pltpu.VMEM((EB,), jnp.int32),
        pltpu.VMEM((EB,), jnp.int32),
        pltpu.SemaphoreType.DMA,
        pltpu.VMEM((EB,), jnp.float32),
        pltpu.VMEM((DCH1,), jnp.float32),
        pltpu.VMEM_SHARED((NP,), jnp.float32),
        pltpu.VMEM_SHARED((NP,), jnp.float32),
    ],
)
def _hist_sc(src_hbm, dst_hbm, zeros_hbm, deg_hbm, cnt_hbm,
             idxs0, idxd0, semi0, idxs1, idxd1, semi1,
             onesb, bounce, accd, accc):
    cid = lax.axis_index("c")
    sid = lax.axis_index("s")
    wid = cid * NS + sid
    bufs = ((idxs0, idxd0, semi0), (idxs1, idxd1, semi1))

    def obody(j, carry):
        onesb[pl.ds(j * 16, 16)] = jnp.ones((16,), jnp.float32)
        return carry

    lax.fori_loop(0, EB // 16, obody, 0)
    _zero_acc(sid, zeros_hbm, accd, DCH1, NCH1)
    _zero_acc(sid, zeros_hbm, accc, DCH1, NCH1)
    plsc.subcore_barrier()

    def valid(i):
        return wid + i * NW < NBLK

    def idx_copies(i, buf):
        base = (wid + i * NW) * EB
        return (pltpu.make_async_copy(src_hbm.at[pl.ds(base, EB)],
                                      buf[0], buf[2]),
                pltpu.make_async_copy(dst_hbm.at[pl.ds(base, EB)],
                                      buf[1], buf[2]))

    @pl.when(valid(0))
    def _():
        c0, c1 = idx_copies(0, bufs[0])
        c0.start()
        c1.start()

    def half(i, bp, bq):
        @pl.when(valid(i))
        def _():
            c0, c1 = idx_copies(i, bp)
            c0.wait()
            c1.wait()

            @pl.when(valid(i + 1))
            def _():
                n0, n1 = idx_copies(i + 1, bq)
                n0.start()
                n1.start()

            pltpu.sync_copy(onesb, accd.at[bp[0]], add=True)
            pltpu.sync_copy(onesb, accc.at[bp[1]], add=True)

    def ebody(j, carry):
        half(2 * j, bufs[0], bufs[1])
        half(2 * j + 1, bufs[1], bufs[0])
        return carry

    lax.fori_loop(0, (WITER + 1) // 2, ebody, 0)
    plsc.subcore_barrier()
    _dump_acc(sid, accd, bounce, deg_hbm.at[cid], DCH1, NCH1)
    _dump_acc(sid, accc, bounce, cnt_hbm.at[cid], DCH1, NCH1)


def _prep_kernel(degp_ref, cntp_ref, x_ref, h_ref,
                 dis_ref, invc_ref, u0x_ref, u0h_ref):
    deg = degp_ref[0, :, 0] + degp_ref[1, :, 0]
    cnt = cntp_ref[0, :, 0] + cntp_ref[1, :, 0]
    dis = jnp.where(deg > 0, lax.rsqrt(jnp.where(deg > 0, deg, 1.0)), 0.0)
    dis_ref[...] = dis[:, None]
    invc_ref[...] = (1.0 / jnp.maximum(cnt, 1.0))[:, None]
    u0x_ref[...] = dis[:, None] * x_ref[...]
    u0h_ref[...] = dis[:, None] * h_ref[...]


def _mid_kernel(s1x_ref, s1h_ref, dis_ref,
                tx1x_ref, u1x_ref, tx1h_ref, u1h_ref):
    dis = dis_ref[...]
    tx1x = -dis * s1x_ref[...]
    tx1h = -dis * s1h_ref[...]
    tx1x_ref[...] = tx1x
    tx1h_ref[...] = tx1h
    u1x_ref[...] = dis * tx1x
    u1h_ref[...] = dis * tx1h


def _gates_kernel(x_ref, h_ref, c_ref, tx1x_ref, s2x_ref, tx1h_ref,
                  s2h_ref, dis_ref, wx_ref, wh_ref, bias_ref, wcs_ref,
                  hn_ref, cn_ref, sy_ref, sy2_ref):
    i = pl.program_id(0)
    dis = dis_ref[...]
    x = x_ref[...]
    h = h_ref[...]
    c = c_ref[...]
    tx2x = -2.0 * dis * s2x_ref[...] - x
    tx2h = -2.0 * dis * s2h_ref[...] - h

    def mm(a, w):
        return jnp.dot(a, w, preferred_element_type=jnp.float32)

    z = (mm(x, wx_ref[0]) + mm(tx1x_ref[...], wx_ref[1]) + mm(tx2x, wx_ref[2])
         + mm(h, wh_ref[0]) + mm(tx1h_ref[...], wh_ref[1]) + mm(tx2h, wh_ref[2])
         + bias_ref[...])
    gi = jax.nn.sigmoid(z[:, 0:F] + wcs_ref[0:1, :] * c)
    gf = jax.nn.sigmoid(z[:, F:2 * F] + wcs_ref[1:2, :] * c)
    gt = jnp.tanh(z[:, 2 * F:3 * F])
    cn = gf * c + gi * gt
    go = jax.nn.sigmoid(z[:, 3 * F:4 * F] + wcs_ref[2:3, :] * cn)
    hn = go * jnp.tanh(cn)
    hn_ref[...] = hn
    cn_ref[...] = cn
    y = jnp.maximum(hn, 0.0)

    @pl.when(i == 0)
    def _():
        sy_ref[...] = jnp.zeros_like(sy_ref)
        sy2_ref[...] = jnp.zeros_like(sy2_ref)

    sy_ref[...] += jnp.sum(y, axis=0, keepdims=True)
    sy2_ref[...] += jnp.sum(y * y, axis=0, keepdims=True)


def _proj_kernel(hn_ref, a_ref, shift_ref, wproj_ref, bproj_ref, wr_ref,
                 xp_ref, yr_ref):
    y = a_ref[...] * jnp.maximum(hn_ref[...], 0.0) + shift_ref[...]
    xp = jnp.dot(y, wproj_ref[...], preferred_element_type=jnp.float32)
    xp_ref[...] = jnp.maximum(xp + bproj_ref[...], 0.0)
    yr_ref[...] = jnp.dot(y, wr_ref[...], preferred_element_type=jnp.float32)


def _out_kernel(sp_ref, invc_ref, yr_ref, wl_ref, bl_ref, out_ref):
    mean_nb = (sp_ref[0] + sp_ref[1]) * invc_ref[...]
    out_ref[...] = (jnp.dot(mean_nb, wl_ref[...],
                            preferred_element_type=jnp.float32)
                    + bl_ref[...] + yr_ref[...])


def _row_spec(width):
    return pl.BlockSpec((ROWS, width), lambda i: (i, 0))


def _part_spec(width):
    return pl.BlockSpec((2, ROWS, width), lambda i: (0, i, 0))


def _full_spec(shape):
    return pl.BlockSpec(shape, lambda i: tuple(0 for _ in shape))


def kernel(x, edge_index, hidden_state, cell_state, edge_weights, params):
    src, dst = edge_index[0], edge_index[1]
    zeros_nf = jnp.zeros((N, F), jnp.float32)
    zeros_np = jnp.zeros((NP,), jnp.float32)

    degp2, cntp2 = _hist_sc(src, dst, zeros_np)
    degp = degp2[:, :N, None]
    cntp = cntp2[:, :N, None]

    dis, invc, u0x, u0h = pl.pallas_call(
        _prep_kernel,
        grid=(GRID,),
        in_specs=[pl.BlockSpec((2, ROWS, 1), lambda i: (0, i, 0)),
                  pl.BlockSpec((2, ROWS, 1), lambda i: (0, i, 0)),
                  _row_spec(F), _row_spec(F)],
        out_specs=[_row_spec(1), _row_spec(1), _row_spec(F), _row_spec(F)],
        out_shape=[jax.ShapeDtypeStruct((N, 1), jnp.float32),
                   jax.ShapeDtypeStruct((N, 1), jnp.float32),
                   jax.ShapeDtypeStruct((N, F), jnp.float32),
                   jax.ShapeDtypeStruct((N, F), jnp.float32)],
    )(degp, cntp, x, hidden_state)

    s1x, s1h = _spmv_pair_sc(u0x, u0h, src, dst, zeros_nf)

    tx1x, u1x, tx1h, u1h = pl.pallas_call(
        _mid_kernel,
        grid=(GRID,),
        in_specs=[_row_spec(F), _row_spec(F), _row_spec(1)],
        out_specs=[_row_spec(F)] * 4,
        out_shape=[jax.ShapeDtypeStruct((N, F), jnp.float32)] * 4,
    )(s1x, s1h, dis)

    s2x, s2h = _spmv_pair_sc(u1x, u1h, src, dst, zeros_nf)

    p = params
    wx = jnp.concatenate([p['Wx_i'], p['Wx_f'], p['Wx_c'], p['Wx_o']], axis=2)
    wh = jnp.concatenate([p['Wh_i'], p['Wh_f'], p['Wh_c'], p['Wh_o']], axis=2)
    bias = jnp.concatenate(
        [p['bx_' + g] + p['bh_' + g] + p['b_' + g][0]
         for g in ('i', 'f', 'c', 'o')]).reshape(1, GATES)
    wcs = jnp.concatenate([p['wc_i'], p['wc_f'], p['wc_o']], axis=0)

    hn, cn, sy, sy2 = pl.pallas_call(
        _gates_kernel,
        grid=(GRID,),
        in_specs=[_row_spec(F), _row_spec(F), _row_spec(F), _row_spec(F),
                  _row_spec(F), _row_spec(F), _row_spec(F), _row_spec(1),
                  _full_spec((3, F, GATES)), _full_spec((3, F, GATES)),
                  _full_spec((1, GATES)), _full_spec((3, F))],
        out_specs=[_row_spec(F), _row_spec(F),
                   pl.BlockSpec((1, F), lambda i: (0, 0)),
                   pl.BlockSpec((1, F), lambda i: (0, 0))],
        out_shape=[jax.ShapeDtypeStruct((N, F), jnp.float32),
                   jax.ShapeDtypeStruct((N, F), jnp.float32),
                   jax.ShapeDtypeStruct((1, F), jnp.float32),
                   jax.ShapeDtypeStruct((1, F), jnp.float32)],
    )(x, hidden_state, cell_state, tx1x, s2x, tx1h, s2h, dis,
      wx, wh, bias, wcs)

    m = sy / N
    m2 = sy2 / N
    gms = p['gn_mean_scale'][None, :]
    var = m2 - 2.0 * gms * m * m + gms * gms * m * m
    a = p['gn_weight'][None, :] * lax.rsqrt(var + 1e-5)
    shift = p['gn_bias'][None, :] - a * gms * m

    xp, yr = pl.pallas_call(
        _proj_kernel,
        grid=(GRID,),
        in_specs=[_row_spec(F), _full_spec((1, F)), _full_spec((1, F)),
                  _full_spec((F, F)), _full_spec((1, F)), _full_spec((F, 1))],
        out_specs=[_row_spec(F), _row_spec(1)],
        out_shape=[jax.ShapeDtypeStruct((N, F), jnp.float32),
                   jax.ShapeDtypeStruct((N, 1), jnp.float32)],
    )(hn, a, shift, p['W_proj'], p['b_proj'][None, :], p['W_r'])

    sp = _spmv_single_sc(xp, src, dst, zeros_nf)

    out = pl.pallas_call(
        _out_kernel,
        grid=(GRID,),
        in_specs=[_part_spec(F), _row_spec(1), _row_spec(1),
                  _full_spec((F, 1)), _full_spec((1, 1))],
        out_specs=_row_spec(1),
        out_shape=jax.ShapeDtypeStruct((N, 1), jnp.float32),
    )(sp, invc, yr, p['W_l'], p['b_l'][None, :])

    return out, hn, cn
